# baseline (speedup 1.0000x reference)
import functools

import jax
import jax.numpy as jnp
from jax import lax
from jax.experimental import pallas as pl
from jax.experimental.pallas import tpu as pltpu
from jax.experimental.pallas import tpu_sc as plsc

N = 10000
E = 320000
D = 128
DE = 16
F2 = 2 * D

NC = 2
NS = 16
NW = NC * NS
EPT = E // NW
CH = 80
NCHUNK = EPT // CH
CH2 = 40
NCHUNK2 = EPT // CH2
N_ACC = 10240
RPT = N_ACC // NS
RZ = 128
NZ = RPT // RZ


def _vproj_body(vf_ref, wsrc_ref, wdst_ref, psrc_ref, pdst_ref):
    vf = vf_ref[...]
    psrc_ref[...] = jnp.dot(vf, wsrc_ref[...], preferred_element_type=jnp.float32)
    pdst_ref[...] = jnp.dot(vf, wdst_ref[...], preferred_element_type=jnp.float32)


def _eproj_body(ef_ref, we_ref, out_ref):
    out_ref[...] = jnp.dot(ef_ref[...], we_ref[...], preferred_element_type=jnp.float32)


def _stats_body(part_ref, gam_ref, bet_ref, out_ref):
    p = part_ref[...]
    s = jnp.sum(p[:, :F2], axis=0)
    q = jnp.sum(p[:, F2:], axis=0)
    mean = s / E
    var = q / E - mean * mean
    scale = gam_ref[...] / jnp.sqrt(var + 1e-5)
    shift = bet_ref[...] - mean * scale
    out_ref[...] = jnp.concatenate(
        [scale[None, :], shift[None, :], jnp.zeros((6, F2), jnp.float32)], axis=0)


def _final_body(parts_ref, vf_ref, wout_ref, out_ref):
    s = parts_ref[0] + parts_ref[1]
    out_ref[...] = jnp.dot(s, wout_ref[...], preferred_element_type=jnp.float32) + vf_ref[...]


def _sc_wid():
    return lax.axis_index("s") * NC + lax.axis_index("c")


def _pass1_body(src_h, dst_h, psrc, pdst, eproj, part, src_v, dst_v, a_v, b_v,
                c_v, acc_v, sem1, sem2):
    wid = _sc_wid()
    zero = jnp.zeros((16,), jnp.float32)
    for i in range(2 * F2 // 16):
        acc_v[pl.ds(16 * i, 16)] = zero

    def chunk(i, carry):
        base = pl.multiple_of(wid * EPT + i * CH, 8)
        pltpu.sync_copy(src_h.at[pl.ds(base, CH)], src_v)
        pltpu.sync_copy(dst_h.at[pl.ds(base, CH)], dst_v)
        cpa = pltpu.async_copy(psrc.at[src_v], a_v, sem1)
        cpb = pltpu.async_copy(pdst.at[dst_v], b_v, sem2)
        pltpu.sync_copy(eproj.at[pl.ds(base, CH)], c_v)
        cpa.wait()
        cpb.wait()

        def edge(e, c2):
            for fg in range(F2 // 16):
                sl = pl.ds(fg * 16, 16)
                x = a_v[e, sl] + b_v[e, sl] + c_v[e, sl]
                acc_v[sl] += x
                acc_v[pl.ds(F2 + fg * 16, 16)] += x * x
            return c2

        return lax.fori_loop(0, CH, edge, carry)

    lax.fori_loop(0, NCHUNK, chunk, 0)
    pltpu.sync_copy(acc_v, part.at[wid])


def _pass2_body(src_h, dst_h, psrc, pdst, eproj, bnp, parts, src_v, dst_v,
                a_v, b_v, c_v, m_v, sc_v, sh_v, accum, sem1, sem2):
    cid = lax.axis_index("c")
    sid = lax.axis_index("s")
    wid = sid * NC + cid
    pltpu.sync_copy(bnp.at[0], sc_v)
    pltpu.sync_copy(bnp.at[1], sh_v)
    zero = jnp.zeros((16,), jnp.float32)

    def zrow(r, c2):
        for fg in range(D // 16):
            m_v[r, pl.ds(fg * 16, 16)] = zero
        return c2

    lax.fori_loop(0, CH2, zrow, 0)
    for k in range(RPT // CH2):
        pltpu.sync_copy(m_v, accum.at[pl.ds(sid * RPT + k * CH2, CH2)])
    plsc.subcore_barrier()

    def chunk(i, carry):
        base = pl.multiple_of(wid * EPT + i * CH2, 8)
        pltpu.sync_copy(src_h.at[pl.ds(base, CH2)], src_v)
        pltpu.sync_copy(dst_h.at[pl.ds(base, CH2)], dst_v)
        cpa = pltpu.async_copy(psrc.at[src_v], a_v, sem1)
        cpb = pltpu.async_copy(pdst.at[dst_v], b_v, sem2)
        pltpu.sync_copy(eproj.at[pl.ds(base, CH2)], c_v)
        cpa.wait()
        cpb.wait()

        def edge(e, c2):
            for p in range(D // 16):
                slc = pl.ds(p * 16, 16)
                slg = pl.ds(D + p * 16, 16)
                xc = (a_v[e, slc] + b_v[e, slc] + c_v[e, slc]) * sc_v[slc] + sh_v[slc]
                xg = (a_v[e, slg] + b_v[e, slg] + c_v[e, slg]) * sc_v[slg] + sh_v[slg]
                s1 = 1.0 / (1.0 + jnp.exp(-xc))
                s2 = 1.0 / (1.0 + jnp.exp(-xg))
                m_v[e, slc] = xc * s1 * s2
            return c2

        lax.fori_loop(0, CH2, edge, carry)
        pltpu.sync_copy(m_v, accum.at[src_v], add=True)
        return carry

    lax.fori_loop(0, NCHUNK2, chunk, 0)
    plsc.subcore_barrier()
    for k in range(NZ):
        row = sid * RPT + k * RZ
        pltpu.sync_copy(accum.at[pl.ds(row, RZ)], parts.at[cid, pl.ds(row, RZ)])


_MESH = plsc.VectorSubcoreMesh(core_axis_name="c", subcore_axis_name="s")

_pass1 = functools.partial(
    pl.kernel,
    mesh=_MESH,
    out_type=jax.ShapeDtypeStruct((NW, 2 * F2), jnp.float32),
    scratch_types=[
        pltpu.VMEM((CH,), jnp.int32),
        pltpu.VMEM((CH,), jnp.int32),
        pltpu.VMEM((CH, F2), jnp.float32),
        pltpu.VMEM((CH, F2), jnp.float32),
        pltpu.VMEM((CH, F2), jnp.float32),
        pltpu.VMEM((2 * F2,), jnp.float32),
        pltpu.SemaphoreType.DMA,
        pltpu.SemaphoreType.DMA,
    ],
)(_pass1_body)

_pass2 = functools.partial(
    pl.kernel,
    mesh=_MESH,
    out_type=jax.ShapeDtypeStruct((NC, N_ACC, D), jnp.float32),
    scratch_types=[
        pltpu.VMEM((CH2,), jnp.int32),
        pltpu.VMEM((CH2,), jnp.int32),
        pltpu.VMEM((CH2, F2), jnp.float32),
        pltpu.VMEM((CH2, F2), jnp.float32),
        pltpu.VMEM((CH2, F2), jnp.float32),
        pltpu.VMEM((CH2, D), jnp.float32),
        pltpu.VMEM((F2,), jnp.float32),
        pltpu.VMEM((F2,), jnp.float32),
        pltpu.VMEM_SHARED((N_ACC, D), jnp.float32),
        pltpu.SemaphoreType.DMA,
        pltpu.SemaphoreType.DMA,
    ],
)(_pass2_body)

_VB = 400
_EB = 1000


def kernel(vertex_feat, edge_feat, edge_index, W_core, W_gate, g_core,
           b_core, g_gate, b_gate, W_out):
    wsrc = jnp.concatenate([W_core[:, :D].T, W_gate[:, :D].T], axis=1)
    wdst = jnp.concatenate([W_core[:, D + DE:].T, W_gate[:, D + DE:].T], axis=1)
    we = jnp.concatenate([W_core[:, D:D + DE].T, W_gate[:, D:D + DE].T], axis=1)
    gamma = jnp.concatenate([g_core, g_gate])
    beta = jnp.concatenate([b_core, b_gate])

    psrc, pdst = pl.pallas_call(
        _vproj_body,
        grid=(N // _VB,),
        in_specs=[
            pl.BlockSpec((_VB, D), lambda i: (i, 0)),
            pl.BlockSpec((D, F2), lambda i: (0, 0)),
            pl.BlockSpec((D, F2), lambda i: (0, 0)),
        ],
        out_specs=[
            pl.BlockSpec((_VB, F2), lambda i: (i, 0)),
            pl.BlockSpec((_VB, F2), lambda i: (i, 0)),
        ],
        out_shape=[
            jax.ShapeDtypeStruct((N, F2), jnp.float32),
            jax.ShapeDtypeStruct((N, F2), jnp.float32),
        ],
    )(vertex_feat, wsrc, wdst)

    eproj = pl.pallas_call(
        _eproj_body,
        grid=(E // _EB,),
        in_specs=[
            pl.BlockSpec((_EB, DE), lambda i: (i, 0)),
            pl.BlockSpec((DE, F2), lambda i: (0, 0)),
        ],
        out_specs=pl.BlockSpec((_EB, F2), lambda i: (i, 0)),
        out_shape=jax.ShapeDtypeStruct((E, F2), jnp.float32),
    )(edge_feat, we)

    src = edge_index[0]
    dst = edge_index[1]
    part = _pass1(src, dst, psrc, pdst, eproj)

    bnp = pl.pallas_call(
        _stats_body,
        in_specs=[
            pl.BlockSpec((NW, 2 * F2), lambda: (0, 0)),
            pl.BlockSpec((F2,), lambda: (0,)),
            pl.BlockSpec((F2,), lambda: (0,)),
        ],
        out_specs=pl.BlockSpec((8, F2), lambda: (0, 0)),
        out_shape=jax.ShapeDtypeStruct((8, F2), jnp.float32),
    )(part, gamma, beta)

    parts = _pass2(src, dst, psrc, pdst, eproj, bnp)[:, :N, :]

    out = pl.pallas_call(
        _final_body,
        grid=(N // _VB,),
        in_specs=[
            pl.BlockSpec((NC, _VB, D), lambda i: (0, i, 0)),
            pl.BlockSpec((_VB, D), lambda i: (i, 0)),
            pl.BlockSpec((D, D), lambda i: (0, 0)),
        ],
        out_specs=pl.BlockSpec((_VB, D), lambda i: (i, 0)),
        out_shape=jax.ShapeDtypeStruct((N, D), jnp.float32),
    )(parts, vertex_feat, W_out.T)

    return out


# speedup vs baseline: 1.2945x; 1.2945x over previous
import functools

import numpy as np
import jax
import jax.numpy as jnp
from jax import lax
from jax.experimental import pallas as pl
from jax.experimental.pallas import tpu as pltpu
from jax.experimental.pallas import tpu_sc as plsc

N = 10000
E = 320000
D = 128
DE = 16
F2 = 2 * D

NC = 2
NS = 16
NW = NC * NS
EPT = E // NW
CH1 = 40
NCH1 = EPT // CH1
CH2 = 16
NCH2 = EPT // CH2
N_ACC = 10240
RPT = N_ACC // NS
RZ = 128
NZ = RPT // RZ


def _vproj_body(vf_ref, wsrc_ref, wdst_ref, psrc_ref, pdst_ref):
    vf = vf_ref[...]
    psrc_ref[...] = jnp.dot(vf, wsrc_ref[...], preferred_element_type=jnp.float32)
    pdst_ref[...] = jnp.dot(vf, wdst_ref[...], preferred_element_type=jnp.float32)


def _eproj_body(ef_ref, we_ref, out_ref):
    out_ref[...] = jnp.dot(ef_ref[...], we_ref[...],
                           preferred_element_type=jnp.float32)


def _stats_body(part_ref, gam_ref, bet_ref, out_ref):
    p = part_ref[...]
    s = jnp.sum(p[:, :F2], axis=0)
    q = jnp.sum(p[:, F2:], axis=0)
    mean = s / E
    var = q / E - mean * mean
    scale = gam_ref[...] / jnp.sqrt(var + 1e-5)
    shift = bet_ref[...] - mean * scale
    out_ref[...] = jnp.concatenate(
        [scale[None, :], shift[None, :], jnp.zeros((6, F2), jnp.float32)], axis=0)


def _final_body(parts_ref, vf_ref, wout_ref, out_ref):
    s = parts_ref[0] + parts_ref[1]
    out_ref[...] = jnp.dot(s, wout_ref[...],
                           preferred_element_type=jnp.float32) + vf_ref[...]


def _pass1_body(src_h, dst_h, psrc, pdst, eproj, part,
                sv0, sv1, dv0, dv1, a0, a1, b0, b1, c0, c1, acc_v,
                sis0, sis1, sid0, sid1, sa0, sa1, sb0, sb1, sc0, sc1):
    svs, dvs = [sv0, sv1], [dv0, dv1]
    abufs, bbufs, cbufs = [a0, a1], [b0, b1], [c0, c1]
    sis, sid = [sis0, sis1], [sid0, sid1]
    sas, sbs, scs = [sa0, sa1], [sb0, sb1], [sc0, sc1]
    wid = lax.axis_index("s") * NC + lax.axis_index("c")
    zero = jnp.zeros((16,), jnp.float32)
    for i in range(2 * F2 // 16):
        acc_v[pl.ds(16 * i, 16)] = zero

    def base(j):
        return pl.multiple_of(wid * EPT + j * CH1, 8)

    def issue_idx(j, t):
        pltpu.async_copy(src_h.at[pl.ds(base(j), CH1)], svs[t], sis[t])
        pltpu.async_copy(dst_h.at[pl.ds(base(j), CH1)], dvs[t], sid[t])

    def wait_idx(j, t):
        pltpu.make_async_copy(src_h.at[pl.ds(base(j), CH1)], svs[t], sis[t]).wait()
        pltpu.make_async_copy(dst_h.at[pl.ds(base(j), CH1)], dvs[t], sid[t]).wait()

    def issue_gath(j, t):
        pltpu.async_copy(psrc.at[svs[t]], abufs[t], sas[t])
        pltpu.async_copy(pdst.at[dvs[t]], bbufs[t], sbs[t])
        pltpu.async_copy(eproj.at[pl.ds(base(j), CH1)], cbufs[t], scs[t])

    def wait_gath(j, t):
        pltpu.make_async_copy(psrc.at[svs[t]], abufs[t], sas[t]).wait()
        pltpu.make_async_copy(pdst.at[dvs[t]], bbufs[t], sbs[t]).wait()
        pltpu.make_async_copy(eproj.at[pl.ds(base(j), CH1)], cbufs[t], scs[t]).wait()

    def compute(t):
        for p in range(F2 // 16):
            sl = pl.ds(16 * p, 16)

            def ebody(e, car, t=t, sl=sl):
                s, q = car
                x = abufs[t][e, sl] + bbufs[t][e, sl] + cbufs[t][e, sl]
                return (s + x, q + x * x)

            s, q = lax.fori_loop(0, CH1, ebody, (zero, zero))
            acc_v[pl.ds(16 * p, 16)] += s
            acc_v[pl.ds(F2 + 16 * p, 16)] += q

    issue_idx(0, 0)
    issue_idx(1, 1)
    wait_idx(0, 0)
    issue_gath(0, 0)

    def step(j2, carry):
        for t in range(2):
            j = 2 * j2 + t

            @pl.when(j < NCH1)
            def _(j=j, t=t):
                wait_gath(j, t)
                compute(t)

                @pl.when(j + 1 < NCH1)
                def _(j=j, t=t):
                    wait_idx(j + 1, 1 - t)
                    issue_gath(j + 1, 1 - t)

                @pl.when(j + 2 < NCH1)
                def _(j=j, t=t):
                    issue_idx(j + 2, t)

        return carry

    lax.fori_loop(0, (NCH1 + 1) // 2, step, 0)
    pltpu.sync_copy(acc_v, part.at[wid])


def _pass2_body(src_h, dst_h, psrc, pdst, eproj, bnp, parts,
                sv0, sv1, sv2, sv3, dv0, dv1, dv2, dv3,
                a0, a1, b0, b1, c0, c1, m0, m1, sc_v, sh_v, accum,
                sis0, sis1, sis2, sis3, sid0, sid1, sid2, sid3,
                sa0, sa1, sb0, sb1, sc0, sc1, sm0, sm1):
    svs, dvs = [sv0, sv1, sv2, sv3], [dv0, dv1, dv2, dv3]
    abufs, bbufs, cbufs = [a0, a1], [b0, b1], [c0, c1]
    mbufs = [m0, m1]
    sis, sid = [sis0, sis1, sis2, sis3], [sid0, sid1, sid2, sid3]
    sas, sbs, scs, sms = [sa0, sa1], [sb0, sb1], [sc0, sc1], [sm0, sm1]
    cid = lax.axis_index("c")
    sid_x = lax.axis_index("s")
    wid = sid_x * NC + cid
    zero = jnp.zeros((16,), jnp.float32)
    pltpu.sync_copy(bnp.at[0], sc_v)
    pltpu.sync_copy(bnp.at[1], sh_v)

    def zrow(r, c2):
        for fg in range(D // 16):
            m0[r, pl.ds(fg * 16, 16)] = zero
        return c2

    lax.fori_loop(0, CH2, zrow, 0)
    for k in range(RPT // CH2):
        pltpu.sync_copy(m0, accum.at[pl.ds(sid_x * RPT + k * CH2, CH2)])
    plsc.subcore_barrier()

    def base(j):
        return pl.multiple_of(wid * EPT + j * CH2, 8)

    def issue_idx(j, t):
        pltpu.async_copy(src_h.at[pl.ds(base(j), CH2)], svs[t], sis[t])
        pltpu.async_copy(dst_h.at[pl.ds(base(j), CH2)], dvs[t], sid[t])

    def wait_idx(j, t):
        pltpu.make_async_copy(src_h.at[pl.ds(base(j), CH2)], svs[t], sis[t]).wait()
        pltpu.make_async_copy(dst_h.at[pl.ds(base(j), CH2)], dvs[t], sid[t]).wait()

    def issue_gath(j, it, t):
        pltpu.async_copy(psrc.at[svs[it]], abufs[t], sas[t])
        pltpu.async_copy(pdst.at[dvs[it]], bbufs[t], sbs[t])
        pltpu.async_copy(eproj.at[pl.ds(base(j), CH2)], cbufs[t], scs[t])

    def wait_gath(j, it, t):
        pltpu.make_async_copy(psrc.at[svs[it]], abufs[t], sas[t]).wait()
        pltpu.make_async_copy(pdst.at[dvs[it]], bbufs[t], sbs[t]).wait()
        pltpu.make_async_copy(eproj.at[pl.ds(base(j), CH2)], cbufs[t], scs[t]).wait()

    def compute(t):
        for p in range(D // 16):
            slc = pl.ds(16 * p, 16)
            slg = pl.ds(D + 16 * p, 16)
            sce = sc_v[slc]
            she = sh_v[slc]
            sge = sc_v[slg]
            hge = sh_v[slg]

            def ebody(e, c2, t=t, slc=slc, slg=slg, sce=sce, she=she,
                      sge=sge, hge=hge):
                xc = (abufs[t][e, slc] + bbufs[t][e, slc]
                      + cbufs[t][e, slc]) * sce + she
                xg = (abufs[t][e, slg] + bbufs[t][e, slg]
                      + cbufs[t][e, slg]) * sge + hge
                mbufs[t][e, slc] = xc / ((1.0 + jnp.exp(-xc))
                                         * (1.0 + jnp.exp(-xg)))
                return c2

            lax.fori_loop(0, CH2, ebody, 0)

    issue_idx(0, 0)
    issue_idx(1, 1)
    wait_idx(0, 0)
    issue_gath(0, 0, 0)

    def step(j2, carry):
        for t in range(4):
            j = 4 * j2 + t
            bt = t % 2

            @pl.when(j < NCH2)
            def _(j=j, t=t, bt=bt):
                @pl.when(j >= 2)
                def _(bt=bt):
                    pltpu.make_async_copy(mbufs[bt], accum.at[svs[t]], sms[bt]).wait()

                wait_gath(j, t, bt)
                compute(bt)
                pltpu.async_copy(mbufs[bt], accum.at[svs[t]], sms[bt], add=True)

                @pl.when(j + 1 < NCH2)
                def _(j=j, t=t, bt=bt):
                    wait_idx(j + 1, (t + 1) % 4)
                    issue_gath(j + 1, (t + 1) % 4, 1 - bt)

                @pl.when(j + 2 < NCH2)
                def _(j=j, t=t):
                    issue_idx(j + 2, (t + 2) % 4)

        return carry

    lax.fori_loop(0, (NCH2 + 3) // 4, step, 0)
    pltpu.make_async_copy(mbufs[0], accum.at[svs[0]], sms[0]).wait()
    pltpu.make_async_copy(mbufs[1], accum.at[svs[1]], sms[1]).wait()
    plsc.subcore_barrier()
    for k in range(NZ):
        row = sid_x * RPT + k * RZ
        pltpu.sync_copy(accum.at[pl.ds(row, RZ)], parts.at[cid, pl.ds(row, RZ)])


_MESH = plsc.VectorSubcoreMesh(core_axis_name="c", subcore_axis_name="s")

_pass1 = functools.partial(
    pl.kernel,
    mesh=_MESH,
    out_type=jax.ShapeDtypeStruct((NW, 2 * F2), jnp.float32),
    scratch_types=(
        [pltpu.VMEM((CH1,), jnp.int32)] * 4
        + [pltpu.VMEM((CH1, F2), jnp.float32)] * 6
        + [pltpu.VMEM((2 * F2,), jnp.float32)]
        + [pltpu.SemaphoreType.DMA] * 10
    ),
)(_pass1_body)

_pass2 = functools.partial(
    pl.kernel,
    mesh=_MESH,
    out_type=jax.ShapeDtypeStruct((NC, N_ACC, D), jnp.float32),
    scratch_types=(
        [pltpu.VMEM((CH2,), jnp.int32)] * 8
        + [pltpu.VMEM((CH2, F2), jnp.float32)] * 6
        + [pltpu.VMEM((CH2, D), jnp.float32)] * 2
        + [pltpu.VMEM((F2,), jnp.float32)] * 2
        + [pltpu.VMEM_SHARED((N_ACC, D), jnp.float32)]
        + [pltpu.SemaphoreType.DMA] * 16
    ),
)(_pass2_body)

_VB = 400
_EB = 1000


def kernel(vertex_feat, edge_feat, edge_index, W_core, W_gate, g_core,
           b_core, g_gate, b_gate, W_out):
    wsrc = jnp.concatenate([W_core[:, :D].T, W_gate[:, :D].T], axis=1)
    wdst = jnp.concatenate([W_core[:, D + DE:].T, W_gate[:, D + DE:].T], axis=1)
    we = jnp.concatenate([W_core[:, D:D + DE].T, W_gate[:, D:D + DE].T], axis=1)
    gamma = jnp.concatenate([g_core, g_gate])
    beta = jnp.concatenate([b_core, b_gate])

    psrc, pdst = pl.pallas_call(
        _vproj_body,
        grid=(N // _VB,),
        in_specs=[
            pl.BlockSpec((_VB, D), lambda i: (i, 0)),
            pl.BlockSpec((D, F2), lambda i: (0, 0)),
            pl.BlockSpec((D, F2), lambda i: (0, 0)),
        ],
        out_specs=[
            pl.BlockSpec((_VB, F2), lambda i: (i, 0)),
            pl.BlockSpec((_VB, F2), lambda i: (i, 0)),
        ],
        out_shape=[
            jax.ShapeDtypeStruct((N, F2), jnp.float32),
            jax.ShapeDtypeStruct((N, F2), jnp.float32),
        ],
    )(vertex_feat, wsrc, wdst)

    eproj = pl.pallas_call(
        _eproj_body,
        grid=(E // _EB,),
        in_specs=[
            pl.BlockSpec((_EB, DE), lambda i: (i, 0)),
            pl.BlockSpec((DE, F2), lambda i: (0, 0)),
        ],
        out_specs=pl.BlockSpec((_EB, F2), lambda i: (i, 0)),
        out_shape=jax.ShapeDtypeStruct((E, F2), jnp.float32),
    )(edge_feat, we)

    src = edge_index[0]
    dst = edge_index[1]
    part = _pass1(src, dst, psrc, pdst, eproj)

    bnp = pl.pallas_call(
        _stats_body,
        in_specs=[
            pl.BlockSpec((NW, 2 * F2), lambda: (0, 0)),
            pl.BlockSpec((F2,), lambda: (0,)),
            pl.BlockSpec((F2,), lambda: (0,)),
        ],
        out_specs=pl.BlockSpec((8, F2), lambda: (0, 0)),
        out_shape=jax.ShapeDtypeStruct((8, F2), jnp.float32),
    )(part, gamma, beta)

    parts = _pass2(src, dst, psrc, pdst, eproj, bnp)[:, :N, :]

    out = pl.pallas_call(
        _final_body,
        grid=(N // _VB,),
        in_specs=[
            pl.BlockSpec((NC, _VB, D), lambda i: (0, i, 0)),
            pl.BlockSpec((_VB, D), lambda i: (i, 0)),
            pl.BlockSpec((D, D), lambda i: (0, 0)),
        ],
        out_specs=pl.BlockSpec((_VB, D), lambda i: (i, 0)),
        out_shape=jax.ShapeDtypeStruct((N, D), jnp.float32),
    )(parts, vertex_feat, W_out.T)

    return out


# speedup vs baseline: 4.3158x; 3.3340x over previous
import functools

import numpy as np
import jax
import jax.numpy as jnp
from jax import lax
from jax.experimental import pallas as pl
from jax.experimental.pallas import tpu as pltpu
from jax.experimental.pallas import tpu_sc as plsc

N = 10000
E = 320000
D = 128
DE = 16
F2 = 2 * D
W2 = F2 // 2

_J = np.concatenate([np.arange(0, F2, 2), np.arange(1, F2, 2)])
_K = np.concatenate([np.arange(0, D, 2), np.arange(1, D, 2)])

NC = 2
NS = 16
NW = NC * NS
EPT = E // NW
CHG = 80
NCHG = EPT // CHG
CHS = 40
NCHS = EPT // CHS
N_ACC = 10240
RPT = N_ACC // NS
RZ = 128
NZ = RPT // RZ


def _vproj_body(vf_ref, wsrc_ref, wdst_ref, psrc_ref, pdst_ref):
    vf = vf_ref[...]
    psrc_ref[...] = jnp.dot(vf, wsrc_ref[...],
                            preferred_element_type=jnp.float32).astype(jnp.bfloat16)
    pdst_ref[...] = jnp.dot(vf, wdst_ref[...],
                            preferred_element_type=jnp.float32).astype(jnp.bfloat16)


def _unpack(w_ref):
    u = lax.bitcast_convert_type(w_ref[...], jnp.uint32)
    lo = lax.bitcast_convert_type(u << jnp.uint32(16), jnp.float32)
    hi = lax.bitcast_convert_type(u & jnp.uint32(0xFFFF0000), jnp.float32)
    return jnp.concatenate([lo, hi], axis=1)


def _stats_body(a_ref, b_ref, ef_ref, we_ref, gam_ref, bet_ref, out_ref, acc_ref):
    i = pl.program_id(0)

    @pl.when(i == 0)
    def _():
        acc_ref[...] = jnp.zeros_like(acc_ref)

    x = (_unpack(a_ref) + _unpack(b_ref)
         + jnp.dot(ef_ref[...], we_ref[...],
                   preferred_element_type=jnp.float32))
    acc_ref[0:1] += jnp.sum(x, axis=0, keepdims=True)
    acc_ref[1:2] += jnp.sum(x * x, axis=0, keepdims=True)

    @pl.when(i == pl.num_programs(0) - 1)
    def _():
        mean = acc_ref[0] / E
        var = acc_ref[1] / E - mean * mean
        scale = gam_ref[...] / jnp.sqrt(var + 1e-5)
        shift = bet_ref[...] - mean * scale
        out_ref[...] = jnp.concatenate(
            [scale[None, :], shift[None, :], jnp.zeros((6, F2), jnp.float32)],
            axis=0)


def _apply_body(a_ref, b_ref, ef_ref, we_ref, bnp_ref, m_ref):
    x = (_unpack(a_ref) + _unpack(b_ref)
         + jnp.dot(ef_ref[...], we_ref[...],
                   preferred_element_type=jnp.float32))
    x = x * bnp_ref[0] + bnp_ref[1]
    h = D // 2
    ce, ge = x[:, :h], x[:, h:D]
    co, go = x[:, D:D + h], x[:, D + h:]
    me = ce * jax.nn.sigmoid(ce) * jax.nn.sigmoid(ge)
    mo = co * jax.nn.sigmoid(co) * jax.nn.sigmoid(go)
    m_ref[...] = jnp.concatenate([me, mo], axis=1)


def _final_body(parts_ref, vf_ref, wout_ref, out_ref):
    s = parts_ref[0] + parts_ref[1]
    out_ref[...] = jnp.dot(s, wout_ref[...],
                           preferred_element_type=jnp.float32) + vf_ref[...]


def _gather_body(src_h, dst_h, psrc, pdst, a_out, b_out,
                 sv0, sv1, dv0, dv1, ab0, ab1, bb0, bb1,
                 sis0, sis1, sid0, sid1, sga0, sga1, sgb0, sgb1,
                 swa0, swa1, swb0, swb1):
    svs, dvs = [sv0, sv1], [dv0, dv1]
    abufs, bbufs = [ab0, ab1], [bb0, bb1]
    sis, sid = [sis0, sis1], [sid0, sid1]
    sga, sgb = [sga0, sga1], [sgb0, sgb1]
    swa, swb = [swa0, swa1], [swb0, swb1]
    wid = lax.axis_index("s") * NC + lax.axis_index("c")

    def base(j):
        return pl.multiple_of(wid * EPT + j * CHG, 8)

    def issue_idx(j, t):
        pltpu.async_copy(src_h.at[pl.ds(base(j), CHG)], svs[t], sis[t])
        pltpu.async_copy(dst_h.at[pl.ds(base(j), CHG)], dvs[t], sid[t])

    def wait_idx(j, t):
        pltpu.make_async_copy(src_h.at[pl.ds(base(j), CHG)], svs[t], sis[t]).wait()
        pltpu.make_async_copy(dst_h.at[pl.ds(base(j), CHG)], dvs[t], sid[t]).wait()

    def issue_g(j, t):
        pltpu.async_copy(psrc.at[svs[t]], abufs[t], sga[t])
        pltpu.async_copy(pdst.at[dvs[t]], bbufs[t], sgb[t])

    def wait_g(j, t):
        pltpu.make_async_copy(psrc.at[svs[t]], abufs[t], sga[t]).wait()
        pltpu.make_async_copy(pdst.at[dvs[t]], bbufs[t], sgb[t]).wait()

    def issue_wr(j, t):
        pltpu.async_copy(abufs[t], a_out.at[pl.ds(base(j), CHG)], swa[t])
        pltpu.async_copy(bbufs[t], b_out.at[pl.ds(base(j), CHG)], swb[t])

    def wait_wr(j, t):
        pltpu.make_async_copy(abufs[t], a_out.at[pl.ds(base(j), CHG)], swa[t]).wait()
        pltpu.make_async_copy(bbufs[t], b_out.at[pl.ds(base(j), CHG)], swb[t]).wait()

    issue_idx(0, 0)
    issue_idx(1, 1)
    wait_idx(0, 0)
    issue_g(0, 0)

    def step(j2, carry):
        for t in range(2):
            j = 2 * j2 + t

            @pl.when(j < NCHG)
            def _(j=j, t=t):
                wait_g(j, t)
                issue_wr(j, t)

                @pl.when(j + 1 < NCHG)
                def _(j=j, t=t):
                    @pl.when(j >= 1)
                    def _(j=j, t=t):
                        wait_wr(j - 1, 1 - t)

                    wait_idx(j + 1, 1 - t)
                    issue_g(j + 1, 1 - t)

                @pl.when(j + 2 < NCHG)
                def _(j=j, t=t):
                    issue_idx(j + 2, t)

        return carry

    lax.fori_loop(0, (NCHG + 1) // 2, step, 0)
    wait_wr(NCHG - 2, (NCHG - 2) % 2)
    wait_wr(NCHG - 1, (NCHG - 1) % 2)


def _scatter_body(src_h, m, parts,
                  sv0, sv1, sv2, sv3, m0, m1, accum,
                  sis0, sis1, sis2, sis3, sr0, sr1, ss0, ss1):
    svs = [sv0, sv1, sv2, sv3]
    mbufs = [m0, m1]
    sis = [sis0, sis1, sis2, sis3]
    srd, ssc = [sr0, sr1], [ss0, ss1]
    cid = lax.axis_index("c")
    sid_x = lax.axis_index("s")
    wid = sid_x * NC + cid
    zero = jnp.zeros((16,), jnp.float32)

    def zrow(r, c2):
        for fg in range(D // 16):
            m0[r, pl.ds(fg * 16, 16)] = zero
        return c2

    lax.fori_loop(0, CHS, zrow, 0)
    for k in range(RPT // CHS):
        pltpu.sync_copy(m0, accum.at[pl.ds(sid_x * RPT + k * CHS, CHS)])
    plsc.subcore_barrier()

    def base(j):
        return pl.multiple_of(wid * EPT + j * CHS, 8)

    def issue_idx(j, it):
        pltpu.async_copy(src_h.at[pl.ds(base(j), CHS)], svs[it], sis[it])

    def wait_idx(j, it):
        pltpu.make_async_copy(src_h.at[pl.ds(base(j), CHS)], svs[it],
                              sis[it]).wait()

    def issue_rd(j, bt):
        pltpu.async_copy(m.at[pl.ds(base(j), CHS)], mbufs[bt], srd[bt])

    def wait_rd(j, bt):
        pltpu.make_async_copy(m.at[pl.ds(base(j), CHS)], mbufs[bt],
                              srd[bt]).wait()

    def issue_scat(it, bt):
        pltpu.async_copy(mbufs[bt], accum.at[svs[it]], ssc[bt], add=True)

    def wait_scat(it, bt):
        pltpu.make_async_copy(mbufs[bt], accum.at[svs[it]], ssc[bt]).wait()

    issue_idx(0, 0)
    issue_idx(1, 1)
    issue_rd(0, 0)

    def step(j2, carry):
        for t in range(4):
            j = 4 * j2 + t
            bt = t % 2

            @pl.when(j < NCHS)
            def _(j=j, t=t, bt=bt):
                wait_rd(j, bt)
                wait_idx(j, t)
                issue_scat(t, bt)

                @pl.when(j + 1 < NCHS)
                def _(j=j, t=t, bt=bt):
                    @pl.when(j >= 1)
                    def _(t=t, bt=bt):
                        wait_scat((t + 3) % 4, 1 - bt)

                    issue_rd(j + 1, 1 - bt)

                @pl.when(j + 2 < NCHS)
                def _(j=j, t=t):
                    issue_idx(j + 2, (t + 2) % 4)

        return carry

    lax.fori_loop(0, (NCHS + 3) // 4, step, 0)
    wait_scat((NCHS - 2) % 4, (NCHS - 2) % 2)
    wait_scat((NCHS - 1) % 4, (NCHS - 1) % 2)
    plsc.subcore_barrier()
    for k in range(NZ):
        row = sid_x * RPT + k * RZ
        pltpu.sync_copy(accum.at[pl.ds(row, RZ)], parts.at[cid, pl.ds(row, RZ)])


_MESH = plsc.VectorSubcoreMesh(core_axis_name="c", subcore_axis_name="s")

_gather = functools.partial(
    pl.kernel,
    mesh=_MESH,
    out_type=(
        jax.ShapeDtypeStruct((E, W2), jnp.float32),
        jax.ShapeDtypeStruct((E, W2), jnp.float32),
    ),
    scratch_types=(
        [pltpu.VMEM((CHG,), jnp.int32)] * 4
        + [pltpu.VMEM((CHG, W2), jnp.float32)] * 4
        + [pltpu.SemaphoreType.DMA] * 12
    ),
)(_gather_body)

_scatter = functools.partial(
    pl.kernel,
    mesh=_MESH,
    out_type=jax.ShapeDtypeStruct((NC, N_ACC, D), jnp.float32),
    scratch_types=(
        [pltpu.VMEM((CHS,), jnp.int32)] * 4
        + [pltpu.VMEM((CHS, D), jnp.float32)] * 2
        + [pltpu.VMEM_SHARED((N_ACC, D), jnp.float32)]
        + [pltpu.SemaphoreType.DMA] * 8
    ),
)(_scatter_body)

_VB = 400
_EB = 2000


def kernel(vertex_feat, edge_feat, edge_index, W_core, W_gate, g_core,
           b_core, g_gate, b_gate, W_out):
    wsrc = jnp.concatenate([W_core[:, :D].T, W_gate[:, :D].T], axis=1)
    wdst = jnp.concatenate([W_core[:, D + DE:].T, W_gate[:, D + DE:].T], axis=1)
    we = jnp.concatenate([W_core[:, D:D + DE].T, W_gate[:, D:D + DE].T],
                         axis=1)[:, _J]
    gamma = jnp.concatenate([g_core, g_gate])[_J]
    beta = jnp.concatenate([b_core, b_gate])[_J]

    psrc, pdst = pl.pallas_call(
        _vproj_body,
        grid=(N // _VB,),
        in_specs=[
            pl.BlockSpec((_VB, D), lambda i: (i, 0)),
            pl.BlockSpec((D, F2), lambda i: (0, 0)),
            pl.BlockSpec((D, F2), lambda i: (0, 0)),
        ],
        out_specs=[
            pl.BlockSpec((_VB, F2), lambda i: (i, 0)),
            pl.BlockSpec((_VB, F2), lambda i: (i, 0)),
        ],
        out_shape=[
            jax.ShapeDtypeStruct((N, F2), jnp.bfloat16),
            jax.ShapeDtypeStruct((N, F2), jnp.bfloat16),
        ],
    )(vertex_feat, wsrc, wdst)

    psrc32 = lax.bitcast_convert_type(psrc.reshape(N, W2, 2), jnp.float32)
    pdst32 = lax.bitcast_convert_type(pdst.reshape(N, W2, 2), jnp.float32)

    src = edge_index[0]
    dst = edge_index[1]
    a, b = _gather(src, dst, psrc32, pdst32)

    bnp = pl.pallas_call(
        _stats_body,
        grid=(E // _EB,),
        in_specs=[
            pl.BlockSpec((_EB, W2), lambda i: (i, 0)),
            pl.BlockSpec((_EB, W2), lambda i: (i, 0)),
            pl.BlockSpec((_EB, DE), lambda i: (i, 0)),
            pl.BlockSpec((DE, F2), lambda i: (0, 0)),
            pl.BlockSpec((F2,), lambda i: (0,)),
            pl.BlockSpec((F2,), lambda i: (0,)),
        ],
        out_specs=pl.BlockSpec((8, F2), lambda i: (0, 0)),
        out_shape=jax.ShapeDtypeStruct((8, F2), jnp.float32),
        scratch_shapes=[pltpu.VMEM((2, F2), jnp.float32)],
    )(a, b, edge_feat, we, gamma, beta)

    m = pl.pallas_call(
        _apply_body,
        grid=(E // _EB,),
        in_specs=[
            pl.BlockSpec((_EB, W2), lambda i: (i, 0)),
            pl.BlockSpec((_EB, W2), lambda i: (i, 0)),
            pl.BlockSpec((_EB, DE), lambda i: (i, 0)),
            pl.BlockSpec((DE, F2), lambda i: (0, 0)),
            pl.BlockSpec((8, F2), lambda i: (0, 0)),
        ],
        out_specs=pl.BlockSpec((_EB, D), lambda i: (i, 0)),
        out_shape=jax.ShapeDtypeStruct((E, D), jnp.float32),
    )(a, b, edge_feat, we, bnp)

    parts = _scatter(src, m)[:, :N, :]

    out = pl.pallas_call(
        _final_body,
        grid=(N // _VB,),
        in_specs=[
            pl.BlockSpec((NC, _VB, D), lambda i: (0, i, 0)),
            pl.BlockSpec((_VB, D), lambda i: (i, 0)),
            pl.BlockSpec((D, D), lambda i: (0, 0)),
        ],
        out_specs=pl.BlockSpec((_VB, D), lambda i: (i, 0)),
        out_shape=jax.ShapeDtypeStruct((N, D), jnp.float32),
    )(parts, vertex_feat, W_out.T[_K])

    return out


# speedup vs baseline: 4.6238x; 1.0714x over previous
import functools

import numpy as np
import jax
import jax.numpy as jnp
from jax import lax
from jax.experimental import pallas as pl
from jax.experimental.pallas import tpu as pltpu
from jax.experimental.pallas import tpu_sc as plsc

N = 10000
E = 320000
D = 128
DE = 16
F2 = 2 * D
W2 = F2 // 2

_J = np.concatenate([np.arange(0, F2, 2), np.arange(1, F2, 2)])
_K = np.concatenate([np.arange(0, D, 2), np.arange(1, D, 2)])

NC = 2
NS = 16
NW = NC * NS
EH = E // 2
EPT = EH // NW
CHG = 40
CHS = 40
N_ACC = 10240
RPT = N_ACC // NS
RZ = 128
NZ = RPT // RZ


def _vproj_body(vf_ref, wsrc_ref, wdst_ref, psrc_ref, pdst_ref):
    vf = vf_ref[...]
    psrc_ref[...] = jnp.dot(vf, wsrc_ref[...],
                            preferred_element_type=jnp.float32).astype(jnp.bfloat16)
    pdst_ref[...] = jnp.dot(vf, wdst_ref[...],
                            preferred_element_type=jnp.float32).astype(jnp.bfloat16)


def _unpack(w_ref):
    u = lax.bitcast_convert_type(w_ref[...], jnp.uint32)
    lo = lax.bitcast_convert_type(u << jnp.uint32(16), jnp.float32)
    hi = lax.bitcast_convert_type(u & jnp.uint32(0xFFFF0000), jnp.float32)
    return jnp.concatenate([lo, hi], axis=1)


def _stats_body(a_ref, b_ref, ef_ref, we_ref, out_ref, acc_ref):
    i = pl.program_id(0)

    @pl.when(i == 0)
    def _():
        acc_ref[...] = jnp.zeros_like(acc_ref)

    x = (_unpack(a_ref) + _unpack(b_ref)
         + jnp.dot(ef_ref[...], we_ref[...],
                   preferred_element_type=jnp.float32))
    acc_ref[0:1] += jnp.sum(x, axis=0, keepdims=True)
    acc_ref[1:2] += jnp.sum(x * x, axis=0, keepdims=True)

    @pl.when(i == pl.num_programs(0) - 1)
    def _():
        out_ref[...] = jnp.concatenate(
            [acc_ref[...], jnp.zeros((6, F2), jnp.float32)], axis=0)


def _bnfold_body(s1_ref, s2_ref, gam_ref, bet_ref, out_ref):
    tot = s1_ref[...] + s2_ref[...]
    mean = tot[0] / E
    var = tot[1] / E - mean * mean
    scale = gam_ref[...] / jnp.sqrt(var + 1e-5)
    shift = bet_ref[...] - mean * scale
    out_ref[...] = jnp.concatenate(
        [scale[None, :], shift[None, :], jnp.zeros((6, F2), jnp.float32)],
        axis=0)


def _apply_body(a_ref, b_ref, ef_ref, we_ref, bnp_ref, m_ref):
    x = (_unpack(a_ref) + _unpack(b_ref)
         + jnp.dot(ef_ref[...], we_ref[...],
                   preferred_element_type=jnp.float32))
    x = x * bnp_ref[0] + bnp_ref[1]
    h = D // 2
    ce, ge = x[:, :h], x[:, h:D]
    co, go = x[:, D:D + h], x[:, D + h:]
    me = ce * jax.nn.sigmoid(ce) * jax.nn.sigmoid(ge)
    mo = co * jax.nn.sigmoid(co) * jax.nn.sigmoid(go)
    m_ref[...] = jnp.concatenate([me, mo], axis=1)


def _final_body(p1_ref, p2_ref, vf_ref, wout_ref, out_ref):
    s = p1_ref[0] + p1_ref[1] + p2_ref[0] + p2_ref[1]
    out_ref[...] = jnp.dot(s, wout_ref[...],
                           preferred_element_type=jnp.float32) + vf_ref[...]


def _make_gather_body(off):
    nchg = EPT // CHG

    def body(src_h, dst_h, psrc, pdst, a_out, b_out,
             sv0, sv1, dv0, dv1, ab0, ab1, bb0, bb1,
             sis0, sis1, sid0, sid1, sga0, sga1, sgb0, sgb1,
             swa0, swa1, swb0, swb1):
        svs, dvs = [sv0, sv1], [dv0, dv1]
        abufs, bbufs = [ab0, ab1], [bb0, bb1]
        sis, sid = [sis0, sis1], [sid0, sid1]
        sga, sgb = [sga0, sga1], [sgb0, sgb1]
        swa, swb = [swa0, swa1], [swb0, swb1]
        wid = lax.axis_index("s") * NC + lax.axis_index("c")

        def obase(j):
            return pl.multiple_of(wid * EPT + j * CHG, 8)

        def ibase(j):
            return pl.multiple_of(off + wid * EPT + j * CHG, 8)

        def issue_idx(j, t):
            pltpu.async_copy(src_h.at[pl.ds(ibase(j), CHG)], svs[t], sis[t])
            pltpu.async_copy(dst_h.at[pl.ds(ibase(j), CHG)], dvs[t], sid[t])

        def wait_idx(j, t):
            pltpu.make_async_copy(src_h.at[pl.ds(ibase(j), CHG)], svs[t],
                                  sis[t]).wait()
            pltpu.make_async_copy(dst_h.at[pl.ds(ibase(j), CHG)], dvs[t],
                                  sid[t]).wait()

        def issue_g(j, t):
            pltpu.async_copy(psrc.at[svs[t]], abufs[t], sga[t])
            pltpu.async_copy(pdst.at[dvs[t]], bbufs[t], sgb[t])

        def wait_g(j, t):
            pltpu.make_async_copy(psrc.at[svs[t]], abufs[t], sga[t]).wait()
            pltpu.make_async_copy(pdst.at[dvs[t]], bbufs[t], sgb[t]).wait()

        def issue_wr(j, t):
            pltpu.async_copy(abufs[t], a_out.at[pl.ds(obase(j), CHG)], swa[t])
            pltpu.async_copy(bbufs[t], b_out.at[pl.ds(obase(j), CHG)], swb[t])

        def wait_wr(j, t):
            pltpu.make_async_copy(abufs[t], a_out.at[pl.ds(obase(j), CHG)],
                                  swa[t]).wait()
            pltpu.make_async_copy(bbufs[t], b_out.at[pl.ds(obase(j), CHG)],
                                  swb[t]).wait()

        issue_idx(0, 0)
        issue_idx(1, 1)
        wait_idx(0, 0)
        issue_g(0, 0)

        def step(j2, carry):
            for t in range(2):
                j = 2 * j2 + t

                @pl.when(j < nchg)
                def _(j=j, t=t):
                    wait_g(j, t)
                    issue_wr(j, t)

                    @pl.when(j + 1 < nchg)
                    def _(j=j, t=t):
                        @pl.when(j >= 1)
                        def _(j=j, t=t):
                            wait_wr(j - 1, 1 - t)

                        wait_idx(j + 1, 1 - t)
                        issue_g(j + 1, 1 - t)

                    @pl.when(j + 2 < nchg)
                    def _(j=j, t=t):
                        issue_idx(j + 2, t)

            return carry

        lax.fori_loop(0, (nchg + 1) // 2, step, 0)
        wait_wr(nchg - 2, (nchg - 2) % 2)
        wait_wr(nchg - 1, (nchg - 1) % 2)

    return body


def _make_scatter_body(off):
    nchs = EPT // CHS

    def body(src_h, m, parts,
             sv0, sv1, sv2, sv3, m0, m1, accum,
             sis0, sis1, sis2, sis3, sr0, sr1, ss0, ss1):
        svs = [sv0, sv1, sv2, sv3]
        mbufs = [m0, m1]
        sis = [sis0, sis1, sis2, sis3]
        srd, ssc = [sr0, sr1], [ss0, ss1]
        cid = lax.axis_index("c")
        sid_x = lax.axis_index("s")
        wid = sid_x * NC + cid
        zero = jnp.zeros((16,), jnp.float32)

        def zrow(r, c2):
            for fg in range(D // 16):
                m0[r, pl.ds(fg * 16, 16)] = zero
            return c2

        lax.fori_loop(0, CHS, zrow, 0)
        for k in range(RPT // CHS):
            pltpu.sync_copy(m0, accum.at[pl.ds(sid_x * RPT + k * CHS, CHS)])
        plsc.subcore_barrier()

        def mbase(j):
            return pl.multiple_of(wid * EPT + j * CHS, 8)

        def ibase(j):
            return pl.multiple_of(off + wid * EPT + j * CHS, 8)

        def issue_idx(j, it):
            pltpu.async_copy(src_h.at[pl.ds(ibase(j), CHS)], svs[it], sis[it])

        def wait_idx(j, it):
            pltpu.make_async_copy(src_h.at[pl.ds(ibase(j), CHS)], svs[it],
                                  sis[it]).wait()

        def issue_rd(j, bt):
            pltpu.async_copy(m.at[pl.ds(mbase(j), CHS)], mbufs[bt], srd[bt])

        def wait_rd(j, bt):
            pltpu.make_async_copy(m.at[pl.ds(mbase(j), CHS)], mbufs[bt],
                                  srd[bt]).wait()

        def issue_scat(it, bt):
            pltpu.async_copy(mbufs[bt], accum.at[svs[it]], ssc[bt], add=True)

        def wait_scat(it, bt):
            pltpu.make_async_copy(mbufs[bt], accum.at[svs[it]], ssc[bt]).wait()

        issue_idx(0, 0)
        issue_idx(1, 1)
        issue_rd(0, 0)

        def step(j2, carry):
            for t in range(4):
                j = 4 * j2 + t
                bt = t % 2

                @pl.when(j < nchs)
                def _(j=j, t=t, bt=bt):
                    wait_rd(j, bt)
                    wait_idx(j, t)
                    issue_scat(t, bt)

                    @pl.when(j + 1 < nchs)
                    def _(j=j, t=t, bt=bt):
                        @pl.when(j >= 1)
                        def _(t=t, bt=bt):
                            wait_scat((t + 3) % 4, 1 - bt)

                        issue_rd(j + 1, 1 - bt)

                    @pl.when(j + 2 < nchs)
                    def _(j=j, t=t):
                        issue_idx(j + 2, (t + 2) % 4)

            return carry

        lax.fori_loop(0, (nchs + 3) // 4, step, 0)
        wait_scat((nchs - 2) % 4, (nchs - 2) % 2)
        wait_scat((nchs - 1) % 4, (nchs - 1) % 2)
        plsc.subcore_barrier()
        for k in range(NZ):
            row = sid_x * RPT + k * RZ
            pltpu.sync_copy(accum.at[pl.ds(row, RZ)],
                            parts.at[cid, pl.ds(row, RZ)])

    return body


_MESH = plsc.VectorSubcoreMesh(core_axis_name="c", subcore_axis_name="s")


def _mk_gather(off):
    return functools.partial(
        pl.kernel,
        mesh=_MESH,
        out_type=(
            jax.ShapeDtypeStruct((EH, W2), jnp.float32),
            jax.ShapeDtypeStruct((EH, W2), jnp.float32),
        ),
        scratch_types=(
            [pltpu.VMEM((CHG,), jnp.int32)] * 4
            + [pltpu.VMEM((CHG, W2), jnp.float32)] * 4
            + [pltpu.SemaphoreType.DMA] * 12
        ),
    )(_make_gather_body(off))


def _mk_scatter(off):
    return functools.partial(
        pl.kernel,
        mesh=_MESH,
        out_type=jax.ShapeDtypeStruct((NC, N_ACC, D), jnp.float32),
        scratch_types=(
            [pltpu.VMEM((CHS,), jnp.int32)] * 4
            + [pltpu.VMEM((CHS, D), jnp.float32)] * 2
            + [pltpu.VMEM_SHARED((N_ACC, D), jnp.float32)]
            + [pltpu.SemaphoreType.DMA] * 8
        ),
    )(_make_scatter_body(off))


_gather1 = _mk_gather(0)
_gather2 = _mk_gather(EH)
_scatter1 = _mk_scatter(0)
_scatter2 = _mk_scatter(EH)

_VB = 400
_EB = 2000
_NEB = EH // _EB


def kernel(vertex_feat, edge_feat, edge_index, W_core, W_gate, g_core,
           b_core, g_gate, b_gate, W_out):
    wsrc = jnp.concatenate([W_core[:, :D].T, W_gate[:, :D].T], axis=1)
    wdst = jnp.concatenate([W_core[:, D + DE:].T, W_gate[:, D + DE:].T], axis=1)
    we = jnp.concatenate([W_core[:, D:D + DE].T, W_gate[:, D:D + DE].T],
                         axis=1)[:, _J]
    gamma = jnp.concatenate([g_core, g_gate])[_J]
    beta = jnp.concatenate([b_core, b_gate])[_J]

    psrc, pdst = pl.pallas_call(
        _vproj_body,
        grid=(N // _VB,),
        in_specs=[
            pl.BlockSpec((_VB, D), lambda i: (i, 0)),
            pl.BlockSpec((D, F2), lambda i: (0, 0)),
            pl.BlockSpec((D, F2), lambda i: (0, 0)),
        ],
        out_specs=[
            pl.BlockSpec((_VB, F2), lambda i: (i, 0)),
            pl.BlockSpec((_VB, F2), lambda i: (i, 0)),
        ],
        out_shape=[
            jax.ShapeDtypeStruct((N, F2), jnp.bfloat16),
            jax.ShapeDtypeStruct((N, F2), jnp.bfloat16),
        ],
    )(vertex_feat, wsrc, wdst)

    psrc32 = lax.bitcast_convert_type(psrc.reshape(N, W2, 2), jnp.float32)
    pdst32 = lax.bitcast_convert_type(pdst.reshape(N, W2, 2), jnp.float32)

    src = edge_index[0]
    dst = edge_index[1]
    a1, b1 = _gather1(src, dst, psrc32, pdst32)
    a2, b2 = _gather2(src, dst, psrc32, pdst32)

    def stats_half(a, b, ef_off):
        return pl.pallas_call(
            _stats_body,
            grid=(_NEB,),
            in_specs=[
                pl.BlockSpec((_EB, W2), lambda i: (i, 0)),
                pl.BlockSpec((_EB, W2), lambda i: (i, 0)),
                pl.BlockSpec((_EB, DE), lambda i, o=ef_off: (i + o, 0)),
                pl.BlockSpec((DE, F2), lambda i: (0, 0)),
            ],
            out_specs=pl.BlockSpec((8, F2), lambda i: (0, 0)),
            out_shape=jax.ShapeDtypeStruct((8, F2), jnp.float32),
            scratch_shapes=[pltpu.VMEM((2, F2), jnp.float32)],
        )(a, b, edge_feat, we)

    s1 = stats_half(a1, b1, 0)
    s2 = stats_half(a2, b2, _NEB)

    bnp = pl.pallas_call(
        _bnfold_body,
        in_specs=[
            pl.BlockSpec((8, F2), lambda: (0, 0)),
            pl.BlockSpec((8, F2), lambda: (0, 0)),
            pl.BlockSpec((F2,), lambda: (0,)),
            pl.BlockSpec((F2,), lambda: (0,)),
        ],
        out_specs=pl.BlockSpec((8, F2), lambda: (0, 0)),
        out_shape=jax.ShapeDtypeStruct((8, F2), jnp.float32),
    )(s1, s2, gamma, beta)

    def apply_half(a, b, ef_off):
        return pl.pallas_call(
            _apply_body,
            grid=(_NEB,),
            in_specs=[
                pl.BlockSpec((_EB, W2), lambda i: (i, 0)),
                pl.BlockSpec((_EB, W2), lambda i: (i, 0)),
                pl.BlockSpec((_EB, DE), lambda i, o=ef_off: (i + o, 0)),
                pl.BlockSpec((DE, F2), lambda i: (0, 0)),
                pl.BlockSpec((8, F2), lambda i: (0, 0)),
            ],
            out_specs=pl.BlockSpec((_EB, D), lambda i: (i, 0)),
            out_shape=jax.ShapeDtypeStruct((EH, D), jnp.float32),
        )(a, b, edge_feat, we, bnp)

    m1 = apply_half(a1, b1, 0)
    m2 = apply_half(a2, b2, _NEB)

    p1 = _scatter1(src, m1)
    p2 = _scatter2(src, m2)

    out = pl.pallas_call(
        _final_body,
        grid=(N // _VB,),
        in_specs=[
            pl.BlockSpec((NC, _VB, D), lambda i: (0, i, 0)),
            pl.BlockSpec((NC, _VB, D), lambda i: (0, i, 0)),
            pl.BlockSpec((_VB, D), lambda i: (i, 0)),
            pl.BlockSpec((D, D), lambda i: (0, 0)),
        ],
        out_specs=pl.BlockSpec((_VB, D), lambda i: (i, 0)),
        out_shape=jax.ShapeDtypeStruct((N, D), jnp.float32),
    )(p1[:, :N, :], p2[:, :N, :], vertex_feat, W_out.T[_K])

    return out


# speedup vs baseline: 4.7566x; 1.0287x over previous
import functools

import numpy as np
import jax
import jax.numpy as jnp
from jax import lax
from jax.experimental import pallas as pl
from jax.experimental.pallas import tpu as pltpu
from jax.experimental.pallas import tpu_sc as plsc

N = 10000
E = 320000
D = 128
DE = 16
F2 = 2 * D
W2 = F2 // 2

_J = np.concatenate([np.arange(0, F2, 2), np.arange(1, F2, 2)])
_K = np.concatenate([np.arange(0, D, 2), np.arange(1, D, 2)])

NC = 2
NS = 16
NW = NC * NS
EH = E // 2
EPT = EH // NW
CHG = 200
CHS = 40
N_ACC = 10240
RPT = N_ACC // NS
RZ = 128
NZ = RPT // RZ


def _vproj_body(vf_ref, wsrc_ref, wdst_ref, psrc_ref, pdst_ref):
    vf = vf_ref[...]
    psrc_ref[...] = jnp.dot(vf, wsrc_ref[...],
                            preferred_element_type=jnp.float32).astype(jnp.bfloat16)
    pdst_ref[...] = jnp.dot(vf, wdst_ref[...],
                            preferred_element_type=jnp.float32).astype(jnp.bfloat16)


def _unpack(w_ref):
    u = lax.bitcast_convert_type(w_ref[...], jnp.uint32)
    lo = lax.bitcast_convert_type(u << jnp.uint32(16), jnp.float32)
    hi = lax.bitcast_convert_type(u & jnp.uint32(0xFFFF0000), jnp.float32)
    return jnp.concatenate([lo, hi], axis=1)


def _stats_body(a_ref, b_ref, ef_ref, we_ref, out_ref, acc_ref):
    i = pl.program_id(0)

    @pl.when(i == 0)
    def _():
        acc_ref[...] = jnp.zeros_like(acc_ref)

    x = (_unpack(a_ref) + _unpack(b_ref)
         + jnp.dot(ef_ref[...], we_ref[...],
                   preferred_element_type=jnp.float32))
    acc_ref[0:1] += jnp.sum(x, axis=0, keepdims=True)
    acc_ref[1:2] += jnp.sum(x * x, axis=0, keepdims=True)

    @pl.when(i == pl.num_programs(0) - 1)
    def _():
        out_ref[...] = jnp.concatenate(
            [acc_ref[...], jnp.zeros((6, F2), jnp.float32)], axis=0)


def _bnfold_body(s1_ref, s2_ref, gam_ref, bet_ref, out_ref):
    tot = s1_ref[...] + s2_ref[...]
    mean = tot[0] / E
    var = tot[1] / E - mean * mean
    scale = gam_ref[...] / jnp.sqrt(var + 1e-5)
    shift = bet_ref[...] - mean * scale
    out_ref[...] = jnp.concatenate(
        [scale[None, :], shift[None, :], jnp.zeros((6, F2), jnp.float32)],
        axis=0)


def _apply_body(a_ref, b_ref, ef_ref, we_ref, bnp_ref, m_ref):
    x = (_unpack(a_ref) + _unpack(b_ref)
         + jnp.dot(ef_ref[...], we_ref[...],
                   preferred_element_type=jnp.float32))
    x = x * bnp_ref[0] + bnp_ref[1]
    h = D // 2
    ce, ge = x[:, :h], x[:, h:D]
    co, go = x[:, D:D + h], x[:, D + h:]
    me = ce * jax.nn.sigmoid(ce) * jax.nn.sigmoid(ge)
    mo = co * jax.nn.sigmoid(co) * jax.nn.sigmoid(go)
    m_ref[...] = jnp.concatenate([me, mo], axis=1)


def _final_body(p1_ref, p2_ref, vf_ref, wout_ref, out_ref):
    s = p1_ref[0] + p1_ref[1] + p2_ref[0] + p2_ref[1]
    out_ref[...] = jnp.dot(s, wout_ref[...],
                           preferred_element_type=jnp.float32) + vf_ref[...]


def _make_gather_body(off):
    nchg = EPT // CHG

    def body(src_h, dst_h, psrc, pdst, a_out, b_out,
             sv0, sv1, dv0, dv1, ab0, ab1, bb0, bb1,
             sis0, sis1, sid0, sid1, sga0, sga1, sgb0, sgb1,
             swa0, swa1, swb0, swb1):
        svs, dvs = [sv0, sv1], [dv0, dv1]
        abufs, bbufs = [ab0, ab1], [bb0, bb1]
        sis, sid = [sis0, sis1], [sid0, sid1]
        sga, sgb = [sga0, sga1], [sgb0, sgb1]
        swa, swb = [swa0, swa1], [swb0, swb1]
        wid = lax.axis_index("s") * NC + lax.axis_index("c")

        def obase(j):
            return pl.multiple_of(wid * EPT + j * CHG, 8)

        def ibase(j):
            return pl.multiple_of(off + wid * EPT + j * CHG, 8)

        def issue_idx(j, t):
            pltpu.async_copy(src_h.at[pl.ds(ibase(j), CHG)], svs[t], sis[t])
            pltpu.async_copy(dst_h.at[pl.ds(ibase(j), CHG)], dvs[t], sid[t])

        def wait_idx(j, t):
            pltpu.make_async_copy(src_h.at[pl.ds(ibase(j), CHG)], svs[t],
                                  sis[t]).wait()
            pltpu.make_async_copy(dst_h.at[pl.ds(ibase(j), CHG)], dvs[t],
                                  sid[t]).wait()

        def issue_g(j, t):
            pltpu.async_copy(psrc.at[svs[t]], abufs[t], sga[t])
            pltpu.async_copy(pdst.at[dvs[t]], bbufs[t], sgb[t])

        def wait_g(j, t):
            pltpu.make_async_copy(psrc.at[svs[t]], abufs[t], sga[t]).wait()
            pltpu.make_async_copy(pdst.at[dvs[t]], bbufs[t], sgb[t]).wait()

        def issue_wr(j, t):
            pltpu.async_copy(abufs[t], a_out.at[pl.ds(obase(j), CHG)], swa[t])
            pltpu.async_copy(bbufs[t], b_out.at[pl.ds(obase(j), CHG)], swb[t])

        def wait_wr(j, t):
            pltpu.make_async_copy(abufs[t], a_out.at[pl.ds(obase(j), CHG)],
                                  swa[t]).wait()
            pltpu.make_async_copy(bbufs[t], b_out.at[pl.ds(obase(j), CHG)],
                                  swb[t]).wait()

        issue_idx(0, 0)
        issue_idx(1, 1)
        wait_idx(0, 0)
        issue_g(0, 0)

        def step(j2, carry):
            for t in range(2):
                j = 2 * j2 + t

                @pl.when(j < nchg)
                def _(j=j, t=t):
                    wait_g(j, t)
                    issue_wr(j, t)

                    @pl.when(j + 1 < nchg)
                    def _(j=j, t=t):
                        @pl.when(j >= 1)
                        def _(j=j, t=t):
                            wait_wr(j - 1, 1 - t)

                        wait_idx(j + 1, 1 - t)
                        issue_g(j + 1, 1 - t)

                    @pl.when(j + 2 < nchg)
                    def _(j=j, t=t):
                        issue_idx(j + 2, t)

            return carry

        lax.fori_loop(0, (nchg + 1) // 2, step, 0)
        wait_wr(nchg - 2, (nchg - 2) % 2)
        wait_wr(nchg - 1, (nchg - 1) % 2)

    return body


def _make_scatter_body(off):
    nchs = EPT // CHS

    def body(src_h, m, parts,
             sv0, sv1, sv2, sv3, m0, m1, accum,
             sis0, sis1, sis2, sis3, sr0, sr1, ss0, ss1):
        svs = [sv0, sv1, sv2, sv3]
        mbufs = [m0, m1]
        sis = [sis0, sis1, sis2, sis3]
        srd, ssc = [sr0, sr1], [ss0, ss1]
        cid = lax.axis_index("c")
        sid_x = lax.axis_index("s")
        wid = sid_x * NC + cid
        zero = jnp.zeros((16,), jnp.float32)

        def zrow(r, c2):
            for fg in range(D // 16):
                m0[r, pl.ds(fg * 16, 16)] = zero
            return c2

        lax.fori_loop(0, CHS, zrow, 0)
        for k in range(RPT // CHS):
            pltpu.sync_copy(m0, accum.at[pl.ds(sid_x * RPT + k * CHS, CHS)])
        plsc.subcore_barrier()

        def mbase(j):
            return pl.multiple_of(wid * EPT + j * CHS, 8)

        def ibase(j):
            return pl.multiple_of(off + wid * EPT + j * CHS, 8)

        def issue_idx(j, it):
            pltpu.async_copy(src_h.at[pl.ds(ibase(j), CHS)], svs[it], sis[it])

        def wait_idx(j, it):
            pltpu.make_async_copy(src_h.at[pl.ds(ibase(j), CHS)], svs[it],
                                  sis[it]).wait()

        def issue_rd(j, bt):
            pltpu.async_copy(m.at[pl.ds(mbase(j), CHS)], mbufs[bt], srd[bt])

        def wait_rd(j, bt):
            pltpu.make_async_copy(m.at[pl.ds(mbase(j), CHS)], mbufs[bt],
                                  srd[bt]).wait()

        def issue_scat(it, bt):
            pltpu.async_copy(mbufs[bt], accum.at[svs[it]], ssc[bt], add=True)

        def wait_scat(it, bt):
            pltpu.make_async_copy(mbufs[bt], accum.at[svs[it]], ssc[bt]).wait()

        issue_idx(0, 0)
        issue_idx(1, 1)
        issue_rd(0, 0)

        def step(j2, carry):
            for t in range(4):
                j = 4 * j2 + t
                bt = t % 2

                @pl.when(j < nchs)
                def _(j=j, t=t, bt=bt):
                    wait_rd(j, bt)
                    wait_idx(j, t)
                    issue_scat(t, bt)

                    @pl.when(j + 1 < nchs)
                    def _(j=j, t=t, bt=bt):
                        @pl.when(j >= 1)
                        def _(t=t, bt=bt):
                            wait_scat((t + 3) % 4, 1 - bt)

                        issue_rd(j + 1, 1 - bt)

                    @pl.when(j + 2 < nchs)
                    def _(j=j, t=t):
                        issue_idx(j + 2, (t + 2) % 4)

            return carry

        lax.fori_loop(0, (nchs + 3) // 4, step, 0)
        wait_scat((nchs - 2) % 4, (nchs - 2) % 2)
        wait_scat((nchs - 1) % 4, (nchs - 1) % 2)
        plsc.subcore_barrier()
        for k in range(NZ):
            row = sid_x * RPT + k * RZ
            pltpu.sync_copy(accum.at[pl.ds(row, RZ)],
                            parts.at[cid, pl.ds(row, RZ)])

    return body


_MESH = plsc.VectorSubcoreMesh(core_axis_name="c", subcore_axis_name="s")


def _mk_gather(off):
    return functools.partial(
        pl.kernel,
        mesh=_MESH,
        out_type=(
            jax.ShapeDtypeStruct((EH, W2), jnp.float32),
            jax.ShapeDtypeStruct((EH, W2), jnp.float32),
        ),
        scratch_types=(
            [pltpu.VMEM((CHG,), jnp.int32)] * 4
            + [pltpu.VMEM((CHG, W2), jnp.float32)] * 4
            + [pltpu.SemaphoreType.DMA] * 12
        ),
    )(_make_gather_body(off))


def _mk_scatter(off):
    return functools.partial(
        pl.kernel,
        mesh=_MESH,
        out_type=jax.ShapeDtypeStruct((NC, N_ACC, D), jnp.float32),
        scratch_types=(
            [pltpu.VMEM((CHS,), jnp.int32)] * 4
            + [pltpu.VMEM((CHS, D), jnp.float32)] * 2
            + [pltpu.VMEM_SHARED((N_ACC, D), jnp.float32)]
            + [pltpu.SemaphoreType.DMA] * 8
        ),
    )(_make_scatter_body(off))


_gather1 = _mk_gather(0)
_gather2 = _mk_gather(EH)
_scatter1 = _mk_scatter(0)
_scatter2 = _mk_scatter(EH)

_VB = 400
_EB = 2000
_NEB = EH // _EB


def kernel(vertex_feat, edge_feat, edge_index, W_core, W_gate, g_core,
           b_core, g_gate, b_gate, W_out):
    wsrc = jnp.concatenate([W_core[:, :D].T, W_gate[:, :D].T], axis=1)
    wdst = jnp.concatenate([W_core[:, D + DE:].T, W_gate[:, D + DE:].T], axis=1)
    we = jnp.concatenate([W_core[:, D:D + DE].T, W_gate[:, D:D + DE].T],
                         axis=1)[:, _J]
    gamma = jnp.concatenate([g_core, g_gate])[_J]
    beta = jnp.concatenate([b_core, b_gate])[_J]

    psrc, pdst = pl.pallas_call(
        _vproj_body,
        grid=(N // _VB,),
        in_specs=[
            pl.BlockSpec((_VB, D), lambda i: (i, 0)),
            pl.BlockSpec((D, F2), lambda i: (0, 0)),
            pl.BlockSpec((D, F2), lambda i: (0, 0)),
        ],
        out_specs=[
            pl.BlockSpec((_VB, F2), lambda i: (i, 0)),
            pl.BlockSpec((_VB, F2), lambda i: (i, 0)),
        ],
        out_shape=[
            jax.ShapeDtypeStruct((N, F2), jnp.bfloat16),
            jax.ShapeDtypeStruct((N, F2), jnp.bfloat16),
        ],
    )(vertex_feat, wsrc, wdst)

    psrc32 = lax.bitcast_convert_type(psrc.reshape(N, W2, 2), jnp.float32)
    pdst32 = lax.bitcast_convert_type(pdst.reshape(N, W2, 2), jnp.float32)

    src = edge_index[0]
    dst = edge_index[1]
    a1, b1 = _gather1(src, dst, psrc32, pdst32)
    a2, b2 = _gather2(src, dst, psrc32, pdst32)

    def stats_half(a, b, ef_off):
        return pl.pallas_call(
            _stats_body,
            grid=(_NEB,),
            in_specs=[
                pl.BlockSpec((_EB, W2), lambda i: (i, 0)),
                pl.BlockSpec((_EB, W2), lambda i: (i, 0)),
                pl.BlockSpec((_EB, DE), lambda i, o=ef_off: (i + o, 0)),
                pl.BlockSpec((DE, F2), lambda i: (0, 0)),
            ],
            out_specs=pl.BlockSpec((8, F2), lambda i: (0, 0)),
            out_shape=jax.ShapeDtypeStruct((8, F2), jnp.float32),
            scratch_shapes=[pltpu.VMEM((2, F2), jnp.float32)],
        )(a, b, edge_feat, we)

    s1 = stats_half(a1, b1, 0)
    s2 = stats_half(a2, b2, _NEB)

    bnp = pl.pallas_call(
        _bnfold_body,
        in_specs=[
            pl.BlockSpec((8, F2), lambda: (0, 0)),
            pl.BlockSpec((8, F2), lambda: (0, 0)),
            pl.BlockSpec((F2,), lambda: (0,)),
            pl.BlockSpec((F2,), lambda: (0,)),
        ],
        out_specs=pl.BlockSpec((8, F2), lambda: (0, 0)),
        out_shape=jax.ShapeDtypeStruct((8, F2), jnp.float32),
    )(s1, s2, gamma, beta)

    def apply_half(a, b, ef_off):
        return pl.pallas_call(
            _apply_body,
            grid=(_NEB,),
            in_specs=[
                pl.BlockSpec((_EB, W2), lambda i: (i, 0)),
                pl.BlockSpec((_EB, W2), lambda i: (i, 0)),
                pl.BlockSpec((_EB, DE), lambda i, o=ef_off: (i + o, 0)),
                pl.BlockSpec((DE, F2), lambda i: (0, 0)),
                pl.BlockSpec((8, F2), lambda i: (0, 0)),
            ],
            out_specs=pl.BlockSpec((_EB, D), lambda i: (i, 0)),
            out_shape=jax.ShapeDtypeStruct((EH, D), jnp.float32),
        )(a, b, edge_feat, we, bnp)

    m1 = apply_half(a1, b1, 0)
    m2 = apply_half(a2, b2, _NEB)

    p1 = _scatter1(src, m1)
    p2 = _scatter2(src, m2)

    out = pl.pallas_call(
        _final_body,
        grid=(N // _VB,),
        in_specs=[
            pl.BlockSpec((NC, _VB, D), lambda i: (0, i, 0)),
            pl.BlockSpec((NC, _VB, D), lambda i: (0, i, 0)),
            pl.BlockSpec((_VB, D), lambda i: (i, 0)),
            pl.BlockSpec((D, D), lambda i: (0, 0)),
        ],
        out_specs=pl.BlockSpec((_VB, D), lambda i: (i, 0)),
        out_shape=jax.ShapeDtypeStruct((N, D), jnp.float32),
    )(p1, p2, vertex_feat, W_out.T[_K])

    return out


# speedup vs baseline: 4.9730x; 1.0455x over previous
import functools

import numpy as np
import jax
import jax.numpy as jnp
from jax import lax
from jax.experimental import pallas as pl
from jax.experimental.pallas import tpu as pltpu
from jax.experimental.pallas import tpu_sc as plsc

N = 10000
E = 320000
D = 128
DE = 16
F2 = 2 * D
W2 = F2 // 2

_J = np.concatenate([np.arange(0, F2, 2), np.arange(1, F2, 2)])
_K = np.concatenate([np.arange(0, D, 2), np.arange(1, D, 2)])

NC = 2
NS = 16
NW = NC * NS
EH = E // 2
EPT = EH // NW
CHG = 200
CHS = 40
N_ACC = 10240
RPT = N_ACC // NS
RZ = 128
NZ = RPT // RZ


def _vproj_body(vf_ref, wsrc_ref, wdst_ref, psrc_ref, pdst_ref):
    vf = vf_ref[...]
    psrc_ref[...] = jnp.dot(vf, wsrc_ref[...],
                            preferred_element_type=jnp.float32).astype(jnp.bfloat16)
    pdst_ref[...] = jnp.dot(vf, wdst_ref[...],
                            preferred_element_type=jnp.float32).astype(jnp.bfloat16)


def _unpack(w_ref):
    u = lax.bitcast_convert_type(w_ref[...], jnp.uint32)
    lo = lax.bitcast_convert_type(u << jnp.uint32(16), jnp.float32)
    hi = lax.bitcast_convert_type(u & jnp.uint32(0xFFFF0000), jnp.float32)
    return jnp.concatenate([lo, hi], axis=1)


def _stats_body(a_ref, b_ref, ef_ref, we_ref, out_ref, acc_ref):
    i = pl.program_id(0)

    @pl.when(i == 0)
    def _():
        acc_ref[...] = jnp.zeros_like(acc_ref)

    x = (_unpack(a_ref) + _unpack(b_ref)
         + jnp.dot(ef_ref[...], we_ref[...],
                   preferred_element_type=jnp.float32))
    acc_ref[0:1] += jnp.sum(x, axis=0, keepdims=True)
    acc_ref[1:2] += jnp.sum(x * x, axis=0, keepdims=True)

    @pl.when(i == pl.num_programs(0) - 1)
    def _():
        out_ref[...] = jnp.concatenate(
            [acc_ref[...], jnp.zeros((6, F2), jnp.float32)], axis=0)


def _bnfold_body(s1_ref, s2_ref, gam_ref, bet_ref, out_ref):
    tot = s1_ref[...] + s2_ref[...]
    mean = tot[0] / E
    var = tot[1] / E - mean * mean
    scale = gam_ref[...] / jnp.sqrt(var + 1e-5)
    shift = bet_ref[...] - mean * scale
    out_ref[...] = jnp.concatenate(
        [scale[None, :], shift[None, :], jnp.zeros((6, F2), jnp.float32)],
        axis=0)


def _apply_body(a_ref, b_ref, ef_ref, we_ref, bnp_ref, m_ref):
    x = (_unpack(a_ref) + _unpack(b_ref)
         + jnp.dot(ef_ref[...], we_ref[...],
                   preferred_element_type=jnp.float32))
    x = x * bnp_ref[0] + bnp_ref[1]
    h = D // 2
    ce, ge = x[:, :h], x[:, h:D]
    co, go = x[:, D:D + h], x[:, D + h:]
    me = ce * jax.nn.sigmoid(ce) * jax.nn.sigmoid(ge)
    mo = co * jax.nn.sigmoid(co) * jax.nn.sigmoid(go)
    m_ref[...] = jnp.concatenate([me, mo], axis=1)


def _final_body(p1_ref, p2_ref, vf_ref, wout_ref, out_ref):
    s = p1_ref[0] + p1_ref[1] + p2_ref[0] + p2_ref[1]
    out_ref[...] = jnp.dot(s, wout_ref[...],
                           preferred_element_type=jnp.float32) + vf_ref[...]


def _make_gather_body(off):
    nchg = EPT // CHG

    def body(src_h, dst_h, psrc, pdst, a_out, b_out,
             sv0, sv1, dv0, dv1, ab0, ab1, bb0, bb1,
             sis0, sis1, sid0, sid1, sga0, sga1, sgb0, sgb1,
             swa0, swa1, swb0, swb1):
        svs, dvs = [sv0, sv1], [dv0, dv1]
        abufs, bbufs = [ab0, ab1], [bb0, bb1]
        sis, sid = [sis0, sis1], [sid0, sid1]
        sga, sgb = [sga0, sga1], [sgb0, sgb1]
        swa, swb = [swa0, swa1], [swb0, swb1]
        wid = lax.axis_index("s") * NC + lax.axis_index("c")

        def obase(j):
            return pl.multiple_of(wid * EPT + j * CHG, 8)

        def ibase(j):
            return pl.multiple_of(off + wid * EPT + j * CHG, 8)

        def issue_idx(j, t):
            pltpu.async_copy(src_h.at[pl.ds(ibase(j), CHG)], svs[t], sis[t])
            pltpu.async_copy(dst_h.at[pl.ds(ibase(j), CHG)], dvs[t], sid[t])

        def wait_idx(j, t):
            pltpu.make_async_copy(src_h.at[pl.ds(ibase(j), CHG)], svs[t],
                                  sis[t]).wait()
            pltpu.make_async_copy(dst_h.at[pl.ds(ibase(j), CHG)], dvs[t],
                                  sid[t]).wait()

        def issue_g(j, t):
            pltpu.async_copy(psrc.at[svs[t]], abufs[t], sga[t])
            pltpu.async_copy(pdst.at[dvs[t]], bbufs[t], sgb[t])

        def wait_g(j, t):
            pltpu.make_async_copy(psrc.at[svs[t]], abufs[t], sga[t]).wait()
            pltpu.make_async_copy(pdst.at[dvs[t]], bbufs[t], sgb[t]).wait()

        def issue_wr(j, t):
            pltpu.async_copy(abufs[t], a_out.at[pl.ds(obase(j), CHG)], swa[t])
            pltpu.async_copy(bbufs[t], b_out.at[pl.ds(obase(j), CHG)], swb[t])

        def wait_wr(j, t):
            pltpu.make_async_copy(abufs[t], a_out.at[pl.ds(obase(j), CHG)],
                                  swa[t]).wait()
            pltpu.make_async_copy(bbufs[t], b_out.at[pl.ds(obase(j), CHG)],
                                  swb[t]).wait()

        issue_idx(0, 0)
        issue_idx(1, 1)
        wait_idx(0, 0)
        issue_g(0, 0)

        def step(j2, carry):
            for t in range(2):
                j = 2 * j2 + t

                @pl.when(j < nchg)
                def _(j=j, t=t):
                    wait_g(j, t)
                    issue_wr(j, t)

                    @pl.when(j + 1 < nchg)
                    def _(j=j, t=t):
                        @pl.when(j >= 1)
                        def _(j=j, t=t):
                            wait_wr(j - 1, 1 - t)

                        wait_idx(j + 1, 1 - t)
                        issue_g(j + 1, 1 - t)

                    @pl.when(j + 2 < nchg)
                    def _(j=j, t=t):
                        issue_idx(j + 2, t)

            return carry

        lax.fori_loop(0, (nchg + 1) // 2, step, 0)
        wait_wr(nchg - 2, (nchg - 2) % 2)
        wait_wr(nchg - 1, (nchg - 1) % 2)

    return body


def _make_scatter_body(off):
    nchs = EPT // CHS

    def body(src_h, m, parts,
             sv0, sv1, sv2, sv3, m0, m1, accum,
             sis0, sis1, sis2, sis3, sr0, sr1, ss0, ss1):
        svs = [sv0, sv1, sv2, sv3]
        mbufs = [m0, m1]
        sis = [sis0, sis1, sis2, sis3]
        srd, ssc = [sr0, sr1], [ss0, ss1]
        cid = lax.axis_index("c")
        sid_x = lax.axis_index("s")
        wid = sid_x * NC + cid
        zero = jnp.zeros((16,), jnp.float32)

        def zrow(r, c2):
            for fg in range(D // 16):
                m0[r, pl.ds(fg * 16, 16)] = zero
            return c2

        lax.fori_loop(0, CHS, zrow, 0)
        for k in range(RPT // CHS):
            pltpu.sync_copy(m0, accum.at[pl.ds(sid_x * RPT + k * CHS, CHS)])
        plsc.subcore_barrier()

        def mbase(j):
            return pl.multiple_of(wid * EPT + j * CHS, 8)

        def ibase(j):
            return pl.multiple_of(off + wid * EPT + j * CHS, 8)

        def issue_idx(j, it):
            pltpu.async_copy(src_h.at[pl.ds(ibase(j), CHS)], svs[it], sis[it])

        def wait_idx(j, it):
            pltpu.make_async_copy(src_h.at[pl.ds(ibase(j), CHS)], svs[it],
                                  sis[it]).wait()

        def issue_rd(j, bt):
            pltpu.async_copy(m.at[pl.ds(mbase(j), CHS)], mbufs[bt], srd[bt])

        def wait_rd(j, bt):
            pltpu.make_async_copy(m.at[pl.ds(mbase(j), CHS)], mbufs[bt],
                                  srd[bt]).wait()

        def issue_scat(it, bt):
            pltpu.async_copy(mbufs[bt], accum.at[svs[it]], ssc[bt], add=True)

        def wait_scat(it, bt):
            pltpu.make_async_copy(mbufs[bt], accum.at[svs[it]], ssc[bt]).wait()

        issue_idx(0, 0)
        issue_idx(1, 1)
        issue_rd(0, 0)

        def step(j2, carry):
            for t in range(4):
                j = 4 * j2 + t
                bt = t % 2

                @pl.when(j < nchs)
                def _(j=j, t=t, bt=bt):
                    wait_rd(j, bt)
                    wait_idx(j, t)
                    issue_scat(t, bt)

                    @pl.when(j + 1 < nchs)
                    def _(j=j, t=t, bt=bt):
                        @pl.when(j >= 1)
                        def _(t=t, bt=bt):
                            wait_scat((t + 3) % 4, 1 - bt)

                        issue_rd(j + 1, 1 - bt)

                    @pl.when(j + 2 < nchs)
                    def _(j=j, t=t):
                        issue_idx(j + 2, (t + 2) % 4)

            return carry

        lax.fori_loop(0, (nchs + 3) // 4, step, 0)
        wait_scat((nchs - 2) % 4, (nchs - 2) % 2)
        wait_scat((nchs - 1) % 4, (nchs - 1) % 2)
        plsc.subcore_barrier()
        for k in range(NZ):
            row = sid_x * RPT + k * RZ
            pltpu.sync_copy(accum.at[pl.ds(row, RZ)],
                            parts.at[cid, pl.ds(row, RZ)])

    return body


_MESH = plsc.VectorSubcoreMesh(core_axis_name="c", subcore_axis_name="s")


def _mk_gather(off):
    return functools.partial(
        pl.kernel,
        mesh=_MESH,
        out_type=(
            jax.ShapeDtypeStruct((EH, W2), jnp.float32),
            jax.ShapeDtypeStruct((EH, W2), jnp.float32),
        ),
        scratch_types=(
            [pltpu.VMEM((CHG,), jnp.int32)] * 4
            + [pltpu.VMEM((CHG, W2), jnp.float32)] * 4
            + [pltpu.SemaphoreType.DMA] * 12
        ),
    )(_make_gather_body(off))


def _mk_scatter(off):
    return functools.partial(
        pl.kernel,
        mesh=_MESH,
        out_type=jax.ShapeDtypeStruct((NC, N_ACC, D), jnp.float32),
        scratch_types=(
            [pltpu.VMEM((CHS,), jnp.int32)] * 4
            + [pltpu.VMEM((CHS, D), jnp.float32)] * 2
            + [pltpu.VMEM_SHARED((N_ACC, D), jnp.float32)]
            + [pltpu.SemaphoreType.DMA] * 8
        ),
    )(_make_scatter_body(off))


_gather1 = _mk_gather(0)
_gather2 = _mk_gather(EH)
_scatter1 = _mk_scatter(0)
_scatter2 = _mk_scatter(EH)

_VB = 400
_EB = 4000
_NEB = EH // _EB


def kernel(vertex_feat, edge_feat, edge_index, W_core, W_gate, g_core,
           b_core, g_gate, b_gate, W_out):
    wsrc = jnp.concatenate([W_core[:, :D].T, W_gate[:, :D].T], axis=1)
    wdst = jnp.concatenate([W_core[:, D + DE:].T, W_gate[:, D + DE:].T], axis=1)
    we = jnp.concatenate([W_core[:, D:D + DE].T, W_gate[:, D:D + DE].T],
                         axis=1)[:, _J]
    gamma = jnp.concatenate([g_core, g_gate])[_J]
    beta = jnp.concatenate([b_core, b_gate])[_J]

    psrc, pdst = pl.pallas_call(
        _vproj_body,
        grid=(N // _VB,),
        in_specs=[
            pl.BlockSpec((_VB, D), lambda i: (i, 0)),
            pl.BlockSpec((D, F2), lambda i: (0, 0)),
            pl.BlockSpec((D, F2), lambda i: (0, 0)),
        ],
        out_specs=[
            pl.BlockSpec((_VB, F2), lambda i: (i, 0)),
            pl.BlockSpec((_VB, F2), lambda i: (i, 0)),
        ],
        out_shape=[
            jax.ShapeDtypeStruct((N, F2), jnp.bfloat16),
            jax.ShapeDtypeStruct((N, F2), jnp.bfloat16),
        ],
    )(vertex_feat, wsrc, wdst)

    psrc32 = lax.bitcast_convert_type(psrc.reshape(N, W2, 2), jnp.float32)
    pdst32 = lax.bitcast_convert_type(pdst.reshape(N, W2, 2), jnp.float32)

    src = edge_index[0]
    dst = edge_index[1]
    a1, b1 = _gather1(src, dst, psrc32, pdst32)
    a2, b2 = _gather2(src, dst, psrc32, pdst32)

    def stats_half(a, b, ef_off):
        return pl.pallas_call(
            _stats_body,
            grid=(_NEB,),
            in_specs=[
                pl.BlockSpec((_EB, W2), lambda i: (i, 0)),
                pl.BlockSpec((_EB, W2), lambda i: (i, 0)),
                pl.BlockSpec((_EB, DE), lambda i, o=ef_off: (i + o, 0)),
                pl.BlockSpec((DE, F2), lambda i: (0, 0)),
            ],
            out_specs=pl.BlockSpec((8, F2), lambda i: (0, 0)),
            out_shape=jax.ShapeDtypeStruct((8, F2), jnp.float32),
            scratch_shapes=[pltpu.VMEM((2, F2), jnp.float32)],
        )(a, b, edge_feat, we)

    s1 = stats_half(a1, b1, 0)
    s2 = stats_half(a2, b2, _NEB)

    bnp = pl.pallas_call(
        _bnfold_body,
        in_specs=[
            pl.BlockSpec((8, F2), lambda: (0, 0)),
            pl.BlockSpec((8, F2), lambda: (0, 0)),
            pl.BlockSpec((F2,), lambda: (0,)),
            pl.BlockSpec((F2,), lambda: (0,)),
        ],
        out_specs=pl.BlockSpec((8, F2), lambda: (0, 0)),
        out_shape=jax.ShapeDtypeStruct((8, F2), jnp.float32),
    )(s1, s2, gamma, beta)

    def apply_half(a, b, ef_off):
        return pl.pallas_call(
            _apply_body,
            grid=(_NEB,),
            in_specs=[
                pl.BlockSpec((_EB, W2), lambda i: (i, 0)),
                pl.BlockSpec((_EB, W2), lambda i: (i, 0)),
                pl.BlockSpec((_EB, DE), lambda i, o=ef_off: (i + o, 0)),
                pl.BlockSpec((DE, F2), lambda i: (0, 0)),
                pl.BlockSpec((8, F2), lambda i: (0, 0)),
            ],
            out_specs=pl.BlockSpec((_EB, D), lambda i: (i, 0)),
            out_shape=jax.ShapeDtypeStruct((EH, D), jnp.float32),
        )(a, b, edge_feat, we, bnp)

    m1 = apply_half(a1, b1, 0)
    m2 = apply_half(a2, b2, _NEB)

    p1 = _scatter1(src, m1)
    p2 = _scatter2(src, m2)

    out = pl.pallas_call(
        _final_body,
        grid=(N // _VB,),
        in_specs=[
            pl.BlockSpec((NC, _VB, D), lambda i: (0, i, 0)),
            pl.BlockSpec((NC, _VB, D), lambda i: (0, i, 0)),
            pl.BlockSpec((_VB, D), lambda i: (i, 0)),
            pl.BlockSpec((D, D), lambda i: (0, 0)),
        ],
        out_specs=pl.BlockSpec((_VB, D), lambda i: (i, 0)),
        out_shape=jax.ShapeDtypeStruct((N, D), jnp.float32),
    )(p1, p2, vertex_feat, W_out.T[_K])

    return out


# speedup vs baseline: 5.0311x; 1.0117x over previous
import functools

import numpy as np
import jax
import jax.numpy as jnp
from jax import lax
from jax.experimental import pallas as pl
from jax.experimental.pallas import tpu as pltpu
from jax.experimental.pallas import tpu_sc as plsc

N = 10000
E = 320000
D = 128
DE = 16
F2 = 2 * D
W2 = F2 // 2

_J = np.concatenate([np.arange(0, F2, 2), np.arange(1, F2, 2)])
_K = np.concatenate([np.arange(0, D, 2), np.arange(1, D, 2)])

NC = 2
NS = 16
NW = NC * NS
EH = E // 2
EPT = EH // NW
CHG = 200
CHS = 40
N_ACC = 10240
RPT = N_ACC // NS
RZ = 128
NZ = RPT // RZ


def _vproj_body(vf_ref, wsrc_ref, wdst_ref, psrc_ref, pdst_ref):
    vf = vf_ref[...]
    psrc_ref[...] = jnp.dot(vf, wsrc_ref[...],
                            preferred_element_type=jnp.float32).astype(jnp.bfloat16)
    pdst_ref[...] = jnp.dot(vf, wdst_ref[...],
                            preferred_element_type=jnp.float32).astype(jnp.bfloat16)


def _unpack(w_ref):
    u = lax.bitcast_convert_type(w_ref[...], jnp.uint32)
    lo = lax.bitcast_convert_type(u << jnp.uint32(16), jnp.float32)
    hi = lax.bitcast_convert_type(u & jnp.uint32(0xFFFF0000), jnp.float32)
    return jnp.concatenate([lo, hi], axis=1)


def _stats_body(a_ref, b_ref, ef_ref, we_ref, out_ref, acc_ref):
    i = pl.program_id(0)

    @pl.when(i == 0)
    def _():
        acc_ref[...] = jnp.zeros_like(acc_ref)

    x = (_unpack(a_ref) + _unpack(b_ref)
         + jnp.dot(ef_ref[...], we_ref[...],
                   preferred_element_type=jnp.float32))
    acc_ref[0:1] += jnp.sum(x, axis=0, keepdims=True)
    acc_ref[1:2] += jnp.sum(x * x, axis=0, keepdims=True)

    @pl.when(i == pl.num_programs(0) - 1)
    def _():
        out_ref[...] = jnp.concatenate(
            [acc_ref[...], jnp.zeros((6, F2), jnp.float32)], axis=0)


def _bnfold_body(s1_ref, s2_ref, gam_ref, bet_ref, out_ref):
    tot = s1_ref[...] + s2_ref[...]
    mean = tot[0] / E
    var = tot[1] / E - mean * mean
    scale = gam_ref[...] / jnp.sqrt(var + 1e-5)
    shift = bet_ref[...] - mean * scale
    out_ref[...] = jnp.concatenate(
        [scale[None, :], shift[None, :], jnp.zeros((6, F2), jnp.float32)],
        axis=0)


def _apply_body(a_ref, b_ref, ef_ref, we_ref, bnp_ref, m_ref):
    x = (_unpack(a_ref) + _unpack(b_ref)
         + jnp.dot(ef_ref[...], we_ref[...],
                   preferred_element_type=jnp.float32))
    x = x * bnp_ref[0] + bnp_ref[1]
    h = D // 2
    ce, ge = x[:, :h], x[:, h:D]
    co, go = x[:, D:D + h], x[:, D + h:]
    me = ce * jax.nn.sigmoid(ce) * jax.nn.sigmoid(ge)
    mo = co * jax.nn.sigmoid(co) * jax.nn.sigmoid(go)
    m_ref[...] = jnp.concatenate([me, mo], axis=1)


def _final_body(p1_ref, p2_ref, vf_ref, wout_ref, out_ref):
    s = p1_ref[0] + p1_ref[1] + p2_ref[0] + p2_ref[1]
    out_ref[...] = jnp.dot(s, wout_ref[...],
                           preferred_element_type=jnp.float32) + vf_ref[...]


def _make_gather_body(off):
    nchg = EPT // CHG

    def body(src_h, dst_h, psrc, pdst, a_out, b_out,
             sv0, sv1, dv0, dv1, ab0, ab1, bb0, bb1,
             sis0, sis1, sid0, sid1, sga0, sga1, sgb0, sgb1,
             swa0, swa1, swb0, swb1):
        svs, dvs = [sv0, sv1], [dv0, dv1]
        abufs, bbufs = [ab0, ab1], [bb0, bb1]
        sis, sid = [sis0, sis1], [sid0, sid1]
        sga, sgb = [sga0, sga1], [sgb0, sgb1]
        swa, swb = [swa0, swa1], [swb0, swb1]
        wid = lax.axis_index("s") * NC + lax.axis_index("c")

        def obase(j):
            return pl.multiple_of(wid * EPT + j * CHG, 8)

        def ibase(j):
            return pl.multiple_of(off + wid * EPT + j * CHG, 8)

        def issue_idx(j, t):
            pltpu.async_copy(src_h.at[pl.ds(ibase(j), CHG)], svs[t], sis[t])
            pltpu.async_copy(dst_h.at[pl.ds(ibase(j), CHG)], dvs[t], sid[t])

        def wait_idx(j, t):
            pltpu.make_async_copy(src_h.at[pl.ds(ibase(j), CHG)], svs[t],
                                  sis[t]).wait()
            pltpu.make_async_copy(dst_h.at[pl.ds(ibase(j), CHG)], dvs[t],
                                  sid[t]).wait()

        def issue_g(j, t):
            pltpu.async_copy(psrc.at[svs[t]], abufs[t], sga[t])
            pltpu.async_copy(pdst.at[dvs[t]], bbufs[t], sgb[t])

        def wait_g(j, t):
            pltpu.make_async_copy(psrc.at[svs[t]], abufs[t], sga[t]).wait()
            pltpu.make_async_copy(pdst.at[dvs[t]], bbufs[t], sgb[t]).wait()

        def issue_wr(j, t):
            pltpu.async_copy(abufs[t], a_out.at[pl.ds(obase(j), CHG)], swa[t])
            pltpu.async_copy(bbufs[t], b_out.at[pl.ds(obase(j), CHG)], swb[t])

        def wait_wr(j, t):
            pltpu.make_async_copy(abufs[t], a_out.at[pl.ds(obase(j), CHG)],
                                  swa[t]).wait()
            pltpu.make_async_copy(bbufs[t], b_out.at[pl.ds(obase(j), CHG)],
                                  swb[t]).wait()

        issue_idx(0, 0)
        issue_idx(1, 1)
        wait_idx(0, 0)
        issue_g(0, 0)

        def step(j2, carry):
            for t in range(2):
                j = 2 * j2 + t

                @pl.when(j < nchg)
                def _(j=j, t=t):
                    wait_g(j, t)
                    issue_wr(j, t)

                    @pl.when(j + 1 < nchg)
                    def _(j=j, t=t):
                        @pl.when(j >= 1)
                        def _(j=j, t=t):
                            wait_wr(j - 1, 1 - t)

                        wait_idx(j + 1, 1 - t)
                        issue_g(j + 1, 1 - t)

                    @pl.when(j + 2 < nchg)
                    def _(j=j, t=t):
                        issue_idx(j + 2, t)

            return carry

        lax.fori_loop(0, (nchg + 1) // 2, step, 0)
        wait_wr(nchg - 2, (nchg - 2) % 2)
        wait_wr(nchg - 1, (nchg - 1) % 2)

    return body


def _make_scatter_body(off):
    nchs = EPT // CHS

    def body(src_h, m, parts,
             sv0, sv1, sv2, sv3, m0, m1, accum,
             sis0, sis1, sis2, sis3, sr0, sr1, ss0, ss1):
        svs = [sv0, sv1, sv2, sv3]
        mbufs = [m0, m1]
        sis = [sis0, sis1, sis2, sis3]
        srd, ssc = [sr0, sr1], [ss0, ss1]
        cid = lax.axis_index("c")
        sid_x = lax.axis_index("s")
        wid = sid_x * NC + cid
        zero = jnp.zeros((16,), jnp.float32)

        def zrow(r, c2):
            for fg in range(D // 16):
                m0[r, pl.ds(fg * 16, 16)] = zero
            return c2

        lax.fori_loop(0, CHS, zrow, 0)
        for k in range(RPT // CHS):
            pltpu.sync_copy(m0, accum.at[pl.ds(sid_x * RPT + k * CHS, CHS)])
        plsc.subcore_barrier()

        def mbase(j):
            return pl.multiple_of(wid * EPT + j * CHS, 8)

        def ibase(j):
            return pl.multiple_of(off + wid * EPT + j * CHS, 8)

        def issue_idx(j, it):
            pltpu.async_copy(src_h.at[pl.ds(ibase(j), CHS)], svs[it], sis[it])

        def wait_idx(j, it):
            pltpu.make_async_copy(src_h.at[pl.ds(ibase(j), CHS)], svs[it],
                                  sis[it]).wait()

        def issue_rd(j, bt):
            pltpu.async_copy(m.at[pl.ds(mbase(j), CHS)], mbufs[bt], srd[bt])

        def wait_rd(j, bt):
            pltpu.make_async_copy(m.at[pl.ds(mbase(j), CHS)], mbufs[bt],
                                  srd[bt]).wait()

        def issue_scat(it, bt):
            pltpu.async_copy(mbufs[bt], accum.at[svs[it]], ssc[bt], add=True)

        def wait_scat(it, bt):
            pltpu.make_async_copy(mbufs[bt], accum.at[svs[it]], ssc[bt]).wait()

        issue_idx(0, 0)
        issue_idx(1, 1)
        issue_rd(0, 0)

        def step(j2, carry):
            for t in range(4):
                j = 4 * j2 + t
                bt = t % 2

                @pl.when(j < nchs)
                def _(j=j, t=t, bt=bt):
                    wait_rd(j, bt)
                    wait_idx(j, t)
                    issue_scat(t, bt)

                    @pl.when(j + 1 < nchs)
                    def _(j=j, t=t, bt=bt):
                        @pl.when(j >= 1)
                        def _(t=t, bt=bt):
                            wait_scat((t + 3) % 4, 1 - bt)

                        issue_rd(j + 1, 1 - bt)

                    @pl.when(j + 2 < nchs)
                    def _(j=j, t=t):
                        issue_idx(j + 2, (t + 2) % 4)

            return carry

        lax.fori_loop(0, (nchs + 3) // 4, step, 0)
        wait_scat((nchs - 2) % 4, (nchs - 2) % 2)
        wait_scat((nchs - 1) % 4, (nchs - 1) % 2)
        plsc.subcore_barrier()
        for k in range(NZ):
            row = sid_x * RPT + k * RZ
            pltpu.sync_copy(accum.at[pl.ds(row, RZ)],
                            parts.at[cid, pl.ds(row, RZ)])

    return body


_MESH = plsc.VectorSubcoreMesh(core_axis_name="c", subcore_axis_name="s")


def _mk_gather(off):
    return functools.partial(
        pl.kernel,
        mesh=_MESH,
        out_type=(
            jax.ShapeDtypeStruct((EH, W2), jnp.float32),
            jax.ShapeDtypeStruct((EH, W2), jnp.float32),
        ),
        scratch_types=(
            [pltpu.VMEM((CHG,), jnp.int32)] * 4
            + [pltpu.VMEM((CHG, W2), jnp.float32)] * 4
            + [pltpu.SemaphoreType.DMA] * 12
        ),
    )(_make_gather_body(off))


def _mk_scatter(off):
    return functools.partial(
        pl.kernel,
        mesh=_MESH,
        out_type=jax.ShapeDtypeStruct((NC, N_ACC, D), jnp.float32),
        scratch_types=(
            [pltpu.VMEM((CHS,), jnp.int32)] * 4
            + [pltpu.VMEM((CHS, D), jnp.float32)] * 2
            + [pltpu.VMEM_SHARED((N_ACC, D), jnp.float32)]
            + [pltpu.SemaphoreType.DMA] * 8
        ),
    )(_make_scatter_body(off))


_gather1 = _mk_gather(0)
_gather2 = _mk_gather(EH)
_scatter1 = _mk_scatter(0)
_scatter2 = _mk_scatter(EH)

_VB = 400
_EB = 8000
_NEB = EH // _EB


def kernel(vertex_feat, edge_feat, edge_index, W_core, W_gate, g_core,
           b_core, g_gate, b_gate, W_out):
    wsrc = jnp.concatenate([W_core[:, :D].T, W_gate[:, :D].T], axis=1)
    wdst = jnp.concatenate([W_core[:, D + DE:].T, W_gate[:, D + DE:].T], axis=1)
    we = jnp.concatenate([W_core[:, D:D + DE].T, W_gate[:, D:D + DE].T],
                         axis=1)[:, _J]
    gamma = jnp.concatenate([g_core, g_gate])[_J]
    beta = jnp.concatenate([b_core, b_gate])[_J]

    psrc, pdst = pl.pallas_call(
        _vproj_body,
        grid=(N // _VB,),
        in_specs=[
            pl.BlockSpec((_VB, D), lambda i: (i, 0)),
            pl.BlockSpec((D, F2), lambda i: (0, 0)),
            pl.BlockSpec((D, F2), lambda i: (0, 0)),
        ],
        out_specs=[
            pl.BlockSpec((_VB, F2), lambda i: (i, 0)),
            pl.BlockSpec((_VB, F2), lambda i: (i, 0)),
        ],
        out_shape=[
            jax.ShapeDtypeStruct((N, F2), jnp.bfloat16),
            jax.ShapeDtypeStruct((N, F2), jnp.bfloat16),
        ],
    )(vertex_feat, wsrc, wdst)

    psrc32 = lax.bitcast_convert_type(psrc.reshape(N, W2, 2), jnp.float32)
    pdst32 = lax.bitcast_convert_type(pdst.reshape(N, W2, 2), jnp.float32)

    src = edge_index[0]
    dst = edge_index[1]
    a1, b1 = _gather1(src, dst, psrc32, pdst32)
    a2, b2 = _gather2(src, dst, psrc32, pdst32)

    def stats_half(a, b, ef_off):
        return pl.pallas_call(
            _stats_body,
            grid=(_NEB,),
            in_specs=[
                pl.BlockSpec((_EB, W2), lambda i: (i, 0)),
                pl.BlockSpec((_EB, W2), lambda i: (i, 0)),
                pl.BlockSpec((_EB, DE), lambda i, o=ef_off: (i + o, 0)),
                pl.BlockSpec((DE, F2), lambda i: (0, 0)),
            ],
            out_specs=pl.BlockSpec((8, F2), lambda i: (0, 0)),
            out_shape=jax.ShapeDtypeStruct((8, F2), jnp.float32),
            scratch_shapes=[pltpu.VMEM((2, F2), jnp.float32)],
        )(a, b, edge_feat, we)

    s1 = stats_half(a1, b1, 0)
    s2 = stats_half(a2, b2, _NEB)

    bnp = pl.pallas_call(
        _bnfold_body,
        in_specs=[
            pl.BlockSpec((8, F2), lambda: (0, 0)),
            pl.BlockSpec((8, F2), lambda: (0, 0)),
            pl.BlockSpec((F2,), lambda: (0,)),
            pl.BlockSpec((F2,), lambda: (0,)),
        ],
        out_specs=pl.BlockSpec((8, F2), lambda: (0, 0)),
        out_shape=jax.ShapeDtypeStruct((8, F2), jnp.float32),
    )(s1, s2, gamma, beta)

    def apply_half(a, b, ef_off):
        return pl.pallas_call(
            _apply_body,
            grid=(_NEB,),
            in_specs=[
                pl.BlockSpec((_EB, W2), lambda i: (i, 0)),
                pl.BlockSpec((_EB, W2), lambda i: (i, 0)),
                pl.BlockSpec((_EB, DE), lambda i, o=ef_off: (i + o, 0)),
                pl.BlockSpec((DE, F2), lambda i: (0, 0)),
                pl.BlockSpec((8, F2), lambda i: (0, 0)),
            ],
            out_specs=pl.BlockSpec((_EB, D), lambda i: (i, 0)),
            out_shape=jax.ShapeDtypeStruct((EH, D), jnp.float32),
        )(a, b, edge_feat, we, bnp)

    m1 = apply_half(a1, b1, 0)
    m2 = apply_half(a2, b2, _NEB)

    p1 = _scatter1(src, m1)
    p2 = _scatter2(src, m2)

    out = pl.pallas_call(
        _final_body,
        grid=(N // _VB,),
        in_specs=[
            pl.BlockSpec((NC, _VB, D), lambda i: (0, i, 0)),
            pl.BlockSpec((NC, _VB, D), lambda i: (0, i, 0)),
            pl.BlockSpec((_VB, D), lambda i: (i, 0)),
            pl.BlockSpec((D, D), lambda i: (0, 0)),
        ],
        out_specs=pl.BlockSpec((_VB, D), lambda i: (i, 0)),
        out_shape=jax.ShapeDtypeStruct((N, D), jnp.float32),
    )(p1, p2, vertex_feat, W_out.T[_K])

    return out
